# Initial kernel scaffold; baseline (speedup 1.0000x reference)
import jax
import jax.numpy as jnp
from jax.experimental import pallas as pl


def kernel(entity_emb, rel_param, W1, a_src1, a_dst1, bias1, W2, a_src2, a_dst2, bias2, layer_emb_W, layer_emb_b, rel_W, rel_b, sub_W, sub_b, out_W, out_b, ln_node_g, ln_node_b, ln_re_g, ln_re_b, b_x, edge_index, b_node_graph_index, sub, rel, shuf_index):
    raise NotImplementedError("write your pallas kernel here")


# speedup vs baseline: 1.0387x; 1.0387x over previous
import functools

import jax
import jax.numpy as jnp
from jax.experimental import pallas as pl
from jax.experimental.pallas import tpu as pltpu

NUM_ENT = 72000
K = 70108
NUM_REL = 14
N_SUB = 70108
E = 400000
B = 4096
NH = 4
FH = 50
D = 200


def _seg_softmax(x, seg, n):
    ex = jnp.exp(x)
    d = jax.ops.segment_sum(ex, seg, num_segments=n)
    return ex / (d[seg] + 1e-16)


def _seg_mean(x, seg, n):
    s = jax.ops.segment_sum(x, seg, num_segments=n)
    c = jax.ops.segment_sum(jnp.ones((x.shape[0], 1), x.dtype), seg, num_segments=n)
    return s / jnp.maximum(c, 1.0)


def _gat_layer(x, src, dst, W, a_s, a_d, b, n):
    proj = (x @ W).reshape(n, NH, FH)
    ss = jnp.sum(proj * a_s[None, :, :], axis=-1)
    st = jnp.sum(proj * a_d[None, :, :], axis=-1)
    e = jax.nn.leaky_relu(ss[src] + st[dst], 0.2)
    att = _seg_softmax(e, dst, n)
    out = jax.ops.segment_sum(proj[src] * att[:, :, None], dst, num_segments=n)
    out = out + x.reshape(n, NH, FH)
    return out.reshape(n, NH * FH) + b


def _ln_kernel(x_ref, g_ref, b_ref, o_ref):
    x = x_ref[...]
    mu = jnp.mean(x, axis=-1, keepdims=True)
    v = jnp.mean((x - mu) ** 2, axis=-1, keepdims=True)
    o_ref[...] = (x - mu) / jnp.sqrt(v + 1e-5) * g_ref[...] + b_ref[...]


def _ln_pallas(x, g, b):
    n, d = x.shape
    blk = 1024
    npad = ((n + blk - 1) // blk) * blk
    xp = jnp.pad(x, ((0, npad - n), (0, 0)))
    out = pl.pallas_call(
        _ln_kernel,
        grid=(npad // blk,),
        in_specs=[
            pl.BlockSpec((blk, d), lambda i: (i, 0)),
            pl.BlockSpec((d,), lambda i: (0,)),
            pl.BlockSpec((d,), lambda i: (0,)),
        ],
        out_specs=pl.BlockSpec((blk, d), lambda i: (i, 0)),
        out_shape=jax.ShapeDtypeStruct((npad, d), x.dtype),
    )(xp, g, b)
    return out[:n]


def kernel(entity_emb, rel_param, W1, a_src1, a_dst1, bias1, W2, a_src2, a_dst2, bias2,
           layer_emb_W, layer_emb_b, rel_W, rel_b, sub_W, sub_b, out_W, out_b,
           ln_node_g, ln_node_b, ln_re_g, ln_re_b,
           b_x, edge_index, b_node_graph_index, sub, rel, shuf_index):
    src, dst = edge_index[0], edge_index[1]
    x = entity_emb[b_x]
    h = _gat_layer(x, src, dst, W1, a_src1, a_dst1, bias1, N_SUB)
    h = jax.nn.elu(h)
    entity_embed = _gat_layer(h, src, dst, W2, a_src2, a_dst2, bias2, N_SUB)
    out = _seg_mean(entity_embed, b_x, K)
    z = out[b_x]
    emb = jnp.concatenate([entity_embed[b_x], z], axis=-1)
    new_emb = emb @ layer_emb_W + layer_emb_b
    z_s = _seg_softmax(new_emb, b_x, K)
    new_out = jax.ops.segment_sum(z_s * emb, b_x, num_segments=K)
    head = new_out @ out_W + out_b + entity_emb[:K]
    new_out = jnp.concatenate([head, entity_emb[K:]], axis=0)
    cnt = jnp.sum(b_node_graph_index[:, None] == jnp.arange(NUM_REL)[None, :], axis=0)
    rel_embeds = jnp.where((cnt > 0)[:, None], rel_param[:NUM_REL], 0.0)
    new_rel = jnp.concatenate([rel_embeds, rel_embeds], axis=0)
    new_rel_out = (new_rel @ rel_W + rel_b) + (rel_param @ rel_W + rel_b)
    entity_con = _ln_pallas(new_out, ln_node_g, ln_node_b)
    rel_con = _ln_pallas(new_rel_out, ln_re_g, ln_re_b)
    sub_rel = emb @ sub_W + sub_b
    se_ = entity_embed[shuf_index]
    sr_ = sub_rel[shuf_index]
    laa = jax.nn.sigmoid(jnp.sum(entity_embed * sub_rel, axis=-1))
    lbb = laa[shuf_index]
    lab = jax.nn.sigmoid(jnp.sum(entity_embed * sr_, axis=-1))
    lba = jax.nn.sigmoid(jnp.sum(se_ * sub_rel, axis=-1))
    cl_loss = jnp.mean(jax.nn.relu(lba - laa + 0.5)) + jnp.mean(jax.nn.relu(lab - lbb + 0.5))
    sub_emb = entity_con[sub]
    rel_emb = rel_con[rel]
    return (sub_emb, rel_emb, entity_con, cl_loss, rel_con)


# speedup vs baseline: 8.7991x; 8.4716x over previous
import functools

import jax
import jax.numpy as jnp
from jax import lax
from jax.experimental import pallas as pl
from jax.experimental.pallas import tpu as pltpu
from jax.experimental.pallas import tpu_sc as plsc

NUM_ENT = 72000
KSEG = 70108
NUM_REL = 14
N_SUB = 70108
E = 400000
BQ = 4096
NH = 4
FH = 50
D = 200

DP = 208
N_PAD = 70656
E_PAD = 400384
NE_PAD = 72192
BLK = 512

_f32 = jnp.float32


def _sc_mesh():
    return plsc.VectorSubcoreMesh(core_axis_name="c", subcore_axis_name="s",
                                  num_cores=2, num_subcores=16)


@functools.partial(jax.jit, static_argnames=("dp", "rb"))
def _sc_gather(table, idx, dp, rb):
    m_tot = idx.shape[0]
    r_pw = m_tot // 32
    nf, rem = divmod(r_pw, rb)

    @functools.partial(
        pl.kernel,
        mesh=_sc_mesh(),
        out_type=jax.ShapeDtypeStruct((m_tot, dp), _f32),
        compiler_params=pltpu.CompilerParams(use_tc_tiling_on_sc=False),
        scratch_types=[
            pltpu.VMEM((rb,), jnp.int32),
            pltpu.VMEM((rb, dp), _f32),
            pltpu.SemaphoreType.DMA,
        ],
    )
    def k(table_hbm, idx_hbm, out_hbm, idx_v, rows_v, sem):
        wid = lax.axis_index("s") * 2 + lax.axis_index("c")
        base0 = wid * r_pw

        def do(base, nb):
            pltpu.sync_copy(idx_hbm.at[pl.ds(base, nb)], idx_v.at[pl.ds(0, nb)])
            pltpu.async_copy(
                table_hbm.at[idx_v.at[pl.ds(0, nb)]],
                rows_v.at[pl.ds(0, nb)], sem).wait()
            pltpu.sync_copy(rows_v.at[pl.ds(0, nb)], out_hbm.at[pl.ds(base, nb)])

        if nf:
            def body(j, _):
                do(base0 + j * rb, rb)
                return 0
            lax.fori_loop(0, nf, body, 0)
        if rem:
            do(base0 + nf * rb, rem)

    return k(table, idx)


T2 = 35328
TRASH = 128


@functools.partial(jax.jit, static_argnames=("t2", "dp", "eb"))
def _sc_scatter_add(vals, idx2, t2, dp, eb):
    m_tot = idx2.shape[0] // 2
    nchunk = dp // 16
    half = (nchunk + 1) // 2
    r_ps = m_tot // 16
    nf, rem = divmod(r_ps, eb)
    tz = t2 // 16
    tzz = (t2 + TRASH) // 16
    zb = min(tzz, 2048)
    znf, zrem = divmod(tzz, zb)

    @functools.partial(
        pl.kernel,
        mesh=_sc_mesh(),
        out_type=jax.ShapeDtypeStruct((2 * t2, dp), _f32),
        compiler_params=pltpu.CompilerParams(use_tc_tiling_on_sc=False),
        scratch_types=[
            pltpu.VMEM((eb,), jnp.int32),
            pltpu.VMEM((eb, 16), _f32),
            pltpu.VMEM((zb, 16), _f32),
            pltpu.VMEM_SHARED((t2 + TRASH, 16), _f32),
        ],
    )
    def k(vals_hbm, idx_hbm, out_hbm, idx_v, val_v, zero_v, acc_sh):
        cid = lax.axis_index("c")
        sid = lax.axis_index("s")

        def zv(i, _):
            zero_v[i] = jnp.zeros((16,), _f32)
            return 0
        lax.fori_loop(0, zb, zv, 0)

        for j in range(half):
            fc = cid * half + j
            for hh in range(2):

                @pl.when(fc < nchunk)
                def _():
                    def zslice(base, nb):
                        pltpu.sync_copy(zero_v.at[pl.ds(0, nb)],
                                        acc_sh.at[pl.ds(base, nb)])
                    row0 = sid * tzz
                    if znf:
                        def zbody(t, _):
                            zslice(row0 + t * zb, zb)
                            return 0
                        lax.fori_loop(0, znf, zbody, 0)
                    if zrem:
                        zslice(row0 + znf * zb, zrem)

                plsc.subcore_barrier()

                @pl.when(fc < nchunk)
                def _():
                    col = fc * 16

                    def scat(base, nb):
                        pltpu.sync_copy(idx_hbm.at[pl.ds(hh * m_tot + base, nb)],
                                        idx_v.at[pl.ds(0, nb)])
                        pltpu.sync_copy(
                            vals_hbm.at[pl.ds(base, nb), pl.ds(col, 16)],
                            val_v.at[pl.ds(0, nb)])
                        pltpu.sync_copy(val_v.at[pl.ds(0, nb)],
                                        acc_sh.at[idx_v.at[pl.ds(0, nb)]],
                                        add=True)

                    base0 = sid * r_ps
                    if nf:
                        def body(t, _):
                            scat(base0 + t * eb, eb)
                            return 0
                        lax.fori_loop(0, nf, body, 0)
                    if rem:
                        scat(base0 + nf * eb, rem)

                plsc.subcore_barrier()

                @pl.when(fc < nchunk)
                def _():
                    col = fc * 16
                    row0 = sid * tz
                    pltpu.sync_copy(
                        acc_sh.at[pl.ds(row0, tz)],
                        out_hbm.at[pl.ds(hh * t2 + row0, tz), pl.ds(col, 16)])

                plsc.subcore_barrier()

    return k(vals, idx2)


def _rows_pc(body, nrows, out_shapes, ins, in_widths):
    grid = (nrows // BLK,)
    in_specs = []
    for a, w in zip(ins, in_widths):
        if w is None:
            nd = a.ndim
            in_specs.append(pl.BlockSpec(a.shape, lambda i, _n=nd: (0,) * _n))
        else:
            in_specs.append(pl.BlockSpec((BLK, w), lambda i: (i, 0)))
    out_specs = [pl.BlockSpec((BLK, s.shape[1]), lambda i: (i, 0))
                 for s in out_shapes]
    if len(out_shapes) == 1:
        out_specs = out_specs[0]
        out_shape = out_shapes[0]
    else:
        out_shape = out_shapes
    return pl.pallas_call(
        body, grid=grid, in_specs=in_specs, out_specs=out_specs,
        out_shape=out_shape)(*ins)


def _k_proj(x_ref, w_ref, as_ref, ad_ref, p_ref, ss_ref, st_ref):
    x = x_ref[...]
    proj = jnp.dot(x, w_ref[...], preferred_element_type=_f32)
    p_ref[...] = proj
    ss_ref[...] = jnp.dot(proj, as_ref[...], preferred_element_type=_f32)
    st_ref[...] = jnp.dot(proj, ad_ref[...], preferred_element_type=_f32)


def _k_scale(nrows_valid, ssrc_ref, stdst_ref, g_ref, eh_ref, sv_ref):
    i = pl.program_id(0)
    s = ssrc_ref[...] + stdst_ref[...]
    ex = jnp.exp(jnp.maximum(s, 0.2 * s))
    rows = i * BLK + lax.broadcasted_iota(jnp.int32, (BLK, 1), 0)
    lanes = lax.broadcasted_iota(jnp.int32, (BLK, 16), 1)
    ex = jnp.where((rows < nrows_valid) & (lanes < NH), ex, 0.0)
    scaled = g_ref[...] * jnp.dot(ex, eh_ref[...], preferred_element_type=_f32)
    sv_ref[...] = jnp.concatenate([scaled, ex], axis=1)


def _k_epi(is_last, nrows_valid, acc_ref, x_ref, b_ref, eh_ref, o_ref):
    acc = acc_ref[...]
    d = jnp.dot(acc[:, DP:], eh_ref[...], preferred_element_type=_f32)
    out = acc[:, :DP] / (d + 1e-16) + x_ref[...] + b_ref[...]
    if not is_last:
        o_ref[...] = jnp.where(out > 0, out, jnp.exp(jnp.minimum(out, 0.0)) - 1.0)
    else:
        i = pl.program_id(0)
        rows = i * BLK + lax.broadcasted_iota(jnp.int32, (BLK, 1), 0)
        cols = lax.broadcasted_iota(jnp.int32, (BLK, DP), 1)
        out = jnp.where(cols == D, 1.0, out)
        o_ref[...] = jnp.where(rows < nrows_valid, out, 0.0)


def _k_idx2(i_ref, lo_ref, hi_ref):
    ix = i_ref[...]
    tr = T2 + (ix & (TRASH - 1))
    lo_ref[...] = jnp.where(ix < T2, ix, tr)
    hi_ref[...] = jnp.where(ix >= T2, ix - T2, tr)


def _mk_idx2(idxp):
    m = idxp.shape[0]
    lo, hi = _rows_pc(
        _k_idx2, m,
        [jax.ShapeDtypeStruct((m, 1), jnp.int32),
         jax.ShapeDtypeStruct((m, 1), jnp.int32)],
        [idxp.reshape(m, 1)], [1])
    return jnp.concatenate([lo.reshape(-1), hi.reshape(-1)])


def _k_out2norm(acc_ref, o_ref):
    acc = acc_ref[...]
    cnt = jnp.maximum(acc[:, D:D + 1], 1.0)
    cols = lax.broadcasted_iota(jnp.int32, (BLK, DP), 1)
    o_ref[...] = jnp.where(cols < D, acc / cnt, 0.0)


def _k_wvals(ee_ref, z_ref, lwa_ref, lwb_ref, lb_ref, wv_ref):
    ee = ee_ref[...]
    z = z_ref[...]
    ne = (jnp.sum(ee * lwa_ref[...], axis=-1, keepdims=True)
          + jnp.sum(z * lwb_ref[...], axis=-1, keepdims=True) + lb_ref[...])
    ex2 = jnp.exp(ne)
    i = pl.program_id(0)
    rows = i * BLK + lax.broadcasted_iota(jnp.int32, (BLK, 1), 0)
    ex2 = jnp.where(rows < N_SUB, ex2, 0.0)
    wv_ref[...] = jnp.concatenate([ee * ex2, z * ex2], axis=1)


def _k_subrel(ee_ref, z_ref, wa_ref, wb_ref, b_ref, o_ref):
    o_ref[...] = (jnp.dot(ee_ref[...], wa_ref[...], preferred_element_type=_f32)
                  + jnp.dot(z_ref[...], wb_ref[...], preferred_element_type=_f32)
                  + b_ref[...])


def _k_head(acc_ref, w_ref, b_ref, ent_ref, o_ref):
    acc = acc_ref[...]
    srow = 1.0 / (acc[:, D:D + 1] + 1e-16)
    o_ref[...] = (jnp.dot(acc, w_ref[...], preferred_element_type=_f32) * srow
                  + b_ref[...] + ent_ref[...])


def _k_ln(x_ref, g_ref, b_ref, o_ref):
    x = x_ref[...]
    mu = jnp.sum(x, axis=-1, keepdims=True) * (1.0 / D)
    v = jnp.sum(x * x, axis=-1, keepdims=True) * (1.0 / D) - mu * mu
    o_ref[...] = (x - mu) * lax.rsqrt(v + 1e-5) * g_ref[...] + b_ref[...]


def _k_loss(ee_ref, sr_ref, see_ref, ssr_ref, o_ref):
    ee = ee_ref[...]
    sr = sr_ref[...]
    se = see_ref[...]
    ss = ssr_ref[...]
    laa = jax.nn.sigmoid(jnp.sum(ee * sr, axis=-1, keepdims=True))
    lbb = jax.nn.sigmoid(jnp.sum(se * ss, axis=-1, keepdims=True))
    lab = jax.nn.sigmoid(jnp.sum(ee * ss, axis=-1, keepdims=True))
    lba = jax.nn.sigmoid(jnp.sum(se * sr, axis=-1, keepdims=True))
    i = pl.program_id(0)
    rows = i * BLK + lax.broadcasted_iota(jnp.int32, (BLK, 1), 0)
    ok = rows < N_SUB
    p0 = jnp.sum(jnp.where(ok, jnp.maximum(lba - laa + 0.5, 0.0), 0.0))
    p1 = jnp.sum(jnp.where(ok, jnp.maximum(lab - lbb + 0.5, 0.0), 0.0))

    @pl.when(i == 0)
    def _():
        o_ref[...] = jnp.zeros((8, 128), _f32)

    lanes = lax.broadcasted_iota(jnp.int32, (8, 128), 1)
    rows8 = lax.broadcasted_iota(jnp.int32, (8, 128), 0)
    o_ref[...] += jnp.where(
        rows8 == 0, jnp.where(lanes == 0, p0, jnp.where(lanes == 1, p1, 0.0)),
        0.0)


def _k_count(b_ref, o_ref):
    i = pl.program_id(0)

    @pl.when(i == 0)
    def _():
        o_ref[...] = jnp.zeros((1, 16), _f32)

    oh = (b_ref[...] == lax.broadcasted_iota(jnp.int32, (BLK, 16), 1))
    o_ref[...] += jnp.sum(oh.astype(_f32), axis=0, keepdims=True)


def _k_relcon(rp_ref, m28_ref, w_ref, b2_ref, g_ref, b_ref, o_ref):
    rp = rp_ref[...]
    rows = lax.broadcasted_iota(jnp.int32, (32, 128), 0)
    low = jnp.where(rows < NUM_REL, rp, 0.0)
    tile14 = low + pltpu.roll(low, NUM_REL, 0)
    nr = tile14 * m28_ref[...] + rp
    y = jnp.dot(nr, w_ref[...], preferred_element_type=_f32) + b2_ref[...]
    mu = jnp.sum(y, axis=-1, keepdims=True) * (1.0 / D)
    v = jnp.sum(y * y, axis=-1, keepdims=True) * (1.0 / D) - mu * mu
    o_ref[...] = (y - mu) * lax.rsqrt(v + 1e-5) * g_ref[...] + b_ref[...]


def _k_relemb(id_ref, rc_ref, o_ref):
    oh = (id_ref[...] == lax.broadcasted_iota(jnp.int32, (BLK, 32), 1))
    o_ref[...] = jnp.dot(oh.astype(_f32), rc_ref[...],
                         preferred_element_type=_f32)


def _pad_rows(a, n):
    return jnp.pad(a, ((0, n - a.shape[0]),) + ((0, 0),) * (a.ndim - 1))


def _pad_idx(a, n):
    return jnp.pad(a.astype(jnp.int32), (0, n - a.shape[0]))


def _padw(w, r, c):
    return jnp.pad(w, ((0, r - w.shape[0]), (0, c - w.shape[1])))


def _gat_layer_pallas(xp, Wp, As16, Ad16, biasp, Eh, srcp, dstp, dst_i2):
    proj, ssT, stT = _rows_pc(
        _k_proj, N_PAD,
        [jax.ShapeDtypeStruct((N_PAD, DP), _f32),
         jax.ShapeDtypeStruct((N_PAD, 16), _f32),
         jax.ShapeDtypeStruct((N_PAD, 16), _f32)],
        [xp, Wp, As16, Ad16], [DP, None, None, None])
    ssrc = _sc_gather(ssT, srcp, 16, 2048)
    stdst = _sc_gather(stT, dstp, 16, 2048)
    g = _sc_gather(proj, srcp, DP, 512)
    sv = _rows_pc(
        functools.partial(_k_scale, E), E_PAD,
        [jax.ShapeDtypeStruct((E_PAD, DP + 16), _f32)],
        [ssrc, stdst, g, Eh], [16, 16, DP, None])
    acc = _sc_scatter_add(sv, dst_i2, T2, DP + 16, 2048)
    return acc


def kernel(entity_emb, rel_param, W1, a_src1, a_dst1, bias1, W2, a_src2, a_dst2, bias2,
           layer_emb_W, layer_emb_b, rel_W, rel_b, sub_W, sub_b, out_W, out_b,
           ln_node_g, ln_node_b, ln_re_g, ln_re_b,
           b_x, edge_index, b_node_graph_index, sub, rel, shuf_index):
    entity_embp = jnp.pad(entity_emb, ((0, 0), (0, DP - D)))
    srcp = _pad_idx(edge_index[0], E_PAD)
    dstp = _pad_idx(edge_index[1], E_PAD)
    b_xp = _pad_idx(b_x, N_PAD)
    shufp = _pad_idx(shuf_index, N_PAD)
    subi = sub.astype(jnp.int32)

    hsel = jnp.repeat(jnp.arange(NH), FH)
    def _mk_a16(a):
        m = jnp.zeros((DP, 16), _f32)
        return m.at[jnp.arange(D), hsel].set(a.reshape(-1))
    As1, Ad1 = _mk_a16(a_src1), _mk_a16(a_dst1)
    As2, Ad2 = _mk_a16(a_src2), _mk_a16(a_dst2)
    Eh = jnp.zeros((16, DP), _f32).at[hsel, jnp.arange(D)].set(1.0)
    W1p = _padw(W1, DP, DP)
    W2p = _padw(W2, DP, DP)
    b1p = _padw(bias1.reshape(1, -1), 1, DP)
    b2p = _padw(bias2.reshape(1, -1), 1, DP)
    lwa = _padw(layer_emb_W[:D].reshape(1, -1), 1, DP)
    lwb = _padw(layer_emb_W[D:].reshape(1, -1), 1, DP)
    lb = layer_emb_b.reshape(1, 1)
    sWa = _padw(sub_W[:D], DP, DP)
    sWb = _padw(sub_W[D:], DP, DP)
    sbp = _padw(sub_b.reshape(1, -1), 1, DP)
    W6 = jnp.zeros((2 * DP, DP), _f32)
    W6 = W6.at[:D, :D].set(out_W[:D]).at[DP:DP + D, :D].set(out_W[D:])
    obp = _padw(out_b.reshape(1, -1), 1, DP)
    lngp = _padw(ln_node_g.reshape(1, -1), 1, DP)
    lnbp = _padw(ln_node_b.reshape(1, -1), 1, DP)
    lngr = _padw(ln_re_g.reshape(1, -1), 1, DP)
    lnbr = _padw(ln_re_b.reshape(1, -1), 1, DP)
    rWp = _padw(rel_W, 100, DP)
    rb2 = _padw((2.0 * rel_b).reshape(1, -1), 1, DP)
    bngi2 = jnp.pad(b_node_graph_index.astype(jnp.int32), (0, N_PAD - N_SUB),
                    constant_values=15).reshape(N_PAD, 1)

    dst_i2 = _mk_idx2(dstp)
    bx_i2 = _mk_idx2(b_xp)

    xp = _sc_gather(entity_embp, b_xp, DP, 512)
    acc1 = _gat_layer_pallas(xp, W1p, As1, Ad1, b1p, Eh, srcp, dstp, dst_i2)
    h = _rows_pc(functools.partial(_k_epi, False, N_SUB), N_PAD,
                 [jax.ShapeDtypeStruct((N_PAD, DP), _f32)],
                 [acc1, xp, b1p, Eh], [DP + 16, DP, None, None])
    acc2 = _gat_layer_pallas(h, W2p, As2, Ad2, b2p, Eh, srcp, dstp, dst_i2)
    eep = _rows_pc(functools.partial(_k_epi, True, N_SUB), N_PAD,
                   [jax.ShapeDtypeStruct((N_PAD, DP), _f32)],
                   [acc2, h, b2p, Eh], [DP + 16, DP, None, None])

    accB = _sc_scatter_add(eep, bx_i2, T2, DP, 2048)
    out2 = _rows_pc(_k_out2norm, N_PAD,
                    [jax.ShapeDtypeStruct((N_PAD, DP), _f32)], [accB], [DP])
    z = _sc_gather(out2, b_xp, DP, 512)
    ee_bx = _sc_gather(eep, b_xp, DP, 512)
    wv = _rows_pc(_k_wvals, N_PAD,
                  [jax.ShapeDtypeStruct((N_PAD, 2 * DP), _f32)],
                  [ee_bx, z, lwa, lwb, lb], [DP, DP, None, None, None])
    acc3 = _sc_scatter_add(wv, bx_i2, T2, 2 * DP, 2048)
    head = _rows_pc(_k_head, N_PAD,
                    [jax.ShapeDtypeStruct((N_PAD, DP), _f32)],
                    [acc3, W6, obp, entity_embp[:N_PAD]],
                    [2 * DP, None, None, DP])
    lnin = _pad_rows(jnp.concatenate([head[:KSEG], entity_embp[KSEG:]], 0),
                     NE_PAD)
    entity_con_p = _rows_pc(_k_ln, NE_PAD,
                            [jax.ShapeDtypeStruct((NE_PAD, DP), _f32)],
                            [lnin, lngp, lnbp], [DP, None, None])
    entity_con = entity_con_p[:NUM_ENT, :D]

    cnt = pl.pallas_call(
        _k_count, grid=(N_PAD // BLK,),
        in_specs=[pl.BlockSpec((BLK, 1), lambda i: (i, 0))],
        out_specs=pl.BlockSpec((1, 16), lambda i: (0, 0)),
        out_shape=jax.ShapeDtypeStruct((1, 16), _f32))(bngi2)
    m14 = (cnt[0, :NUM_REL] > 0).astype(_f32)
    m28 = jnp.concatenate([m14, m14, jnp.zeros((4,), _f32)]).reshape(32, 1)
    rp32 = jnp.pad(rel_param, ((0, 4), (0, 28)))
    rW128 = jnp.pad(rWp, ((0, 28), (0, 0)))
    rc32 = pl.pallas_call(
        _k_relcon, grid=(1,),
        in_specs=[pl.BlockSpec((32, 128), lambda i: (0, 0)),
                  pl.BlockSpec((32, 1), lambda i: (0, 0)),
                  pl.BlockSpec((128, DP), lambda i: (0, 0)),
                  pl.BlockSpec((1, DP), lambda i: (0, 0)),
                  pl.BlockSpec((1, DP), lambda i: (0, 0)),
                  pl.BlockSpec((1, DP), lambda i: (0, 0))],
        out_specs=pl.BlockSpec((32, DP), lambda i: (0, 0)),
        out_shape=jax.ShapeDtypeStruct((32, DP), _f32))(
            rp32, m28, rW128, rb2, lngr, lnbr)
    rel_con = rc32[:2 * NUM_REL, :D]
    rel_emb = _rows_pc(_k_relemb, BQ,
                       [jax.ShapeDtypeStruct((BQ, DP), _f32)],
                       [rel.astype(jnp.int32).reshape(BQ, 1), rc32],
                       [1, None])[:, :D]

    srl = _rows_pc(_k_subrel, N_PAD,
                   [jax.ShapeDtypeStruct((N_PAD, DP), _f32)],
                   [ee_bx, z, sWa, sWb, sbp], [DP, DP, None, None, None])
    se_ = _sc_gather(eep, shufp, DP, 512)
    sr_ = _sc_gather(srl, shufp, DP, 512)
    parts = pl.pallas_call(
        _k_loss, grid=(N_PAD // BLK,),
        in_specs=[pl.BlockSpec((BLK, DP), lambda i: (i, 0))] * 4,
        out_specs=pl.BlockSpec((8, 128), lambda i: (0, 0)),
        out_shape=jax.ShapeDtypeStruct((8, 128), _f32))(
            eep, srl, se_, sr_)
    cl_loss = (parts[0, 0] + parts[0, 1]) / N_SUB

    sub_emb = _sc_gather(entity_con_p, subi, DP, 512)[:, :D]
    return (sub_emb, rel_emb, entity_con, cl_loss, rel_con)


# speedup vs baseline: 10.5177x; 1.1953x over previous
import functools

import jax
import jax.numpy as jnp
from jax import lax
from jax.experimental import pallas as pl
from jax.experimental.pallas import tpu as pltpu
from jax.experimental.pallas import tpu_sc as plsc

NUM_ENT = 72000
KSEG = 70108
NUM_REL = 14
N_SUB = 70108
E = 400000
BQ = 4096
NH = 4
FH = 50
D = 200

DP = 208
DG = 256
N_PAD = 70656
E_PAD = 400384
NE_PAD = 72192
BLK = 512

_f32 = jnp.float32


def _sc_mesh():
    return plsc.VectorSubcoreMesh(core_axis_name="c", subcore_axis_name="s",
                                  num_cores=2, num_subcores=16)


@functools.partial(jax.jit, static_argnames=("dp", "rb", "tiled"))
def _sc_gather(table, idx, dp, rb, tiled=False):
    m_tot = idx.shape[0]
    r_pw = m_tot // 32
    nf, rem = divmod(r_pw, rb)

    @functools.partial(
        pl.kernel,
        mesh=_sc_mesh(),
        out_type=jax.ShapeDtypeStruct((m_tot, dp), _f32),
        compiler_params=pltpu.CompilerParams(use_tc_tiling_on_sc=tiled),
        scratch_types=[
            pltpu.VMEM((rb,), jnp.int32),
            pltpu.VMEM((rb, dp), _f32),
            pltpu.SemaphoreType.DMA,
        ],
    )
    def k(table_hbm, idx_hbm, out_hbm, idx_v, rows_v, sem):
        wid = lax.axis_index("s") * 2 + lax.axis_index("c")
        base0 = wid * r_pw

        def do(base, nb):
            pltpu.sync_copy(idx_hbm.at[pl.ds(base, nb)], idx_v.at[pl.ds(0, nb)])
            pltpu.async_copy(
                table_hbm.at[idx_v.at[pl.ds(0, nb)]],
                rows_v.at[pl.ds(0, nb)], sem).wait()
            pltpu.sync_copy(rows_v.at[pl.ds(0, nb)], out_hbm.at[pl.ds(base, nb)])

        if nf:
            def body(j, _):
                do(base0 + j * rb, rb)
                return 0
            lax.fori_loop(0, nf, body, 0)
        if rem:
            do(base0 + nf * rb, rem)

    return k(table, idx)


T2 = 35328
TRASH = 128


@functools.partial(jax.jit, static_argnames=("t2", "dp", "eb"))
def _sc_scatter_add(vals, idx2, t2, dp, eb):
    m_tot = idx2.shape[0] // 2
    nchunk = dp // 16
    half = (nchunk + 1) // 2
    r_ps = m_tot // 16
    nf, rem = divmod(r_ps, eb)
    tz = t2 // 16
    tzz = (t2 + TRASH) // 16
    zb = min(tzz, 2048)
    znf, zrem = divmod(tzz, zb)

    @functools.partial(
        pl.kernel,
        mesh=_sc_mesh(),
        out_type=jax.ShapeDtypeStruct((2 * t2, dp), _f32),
        compiler_params=pltpu.CompilerParams(use_tc_tiling_on_sc=False),
        scratch_types=[
            pltpu.VMEM((eb,), jnp.int32),
            pltpu.VMEM((eb, 16), _f32),
            pltpu.VMEM((zb, 16), _f32),
            pltpu.VMEM_SHARED((t2 + TRASH, 16), _f32),
        ],
    )
    def k(vals_hbm, idx_hbm, out_hbm, idx_v, val_v, zero_v, acc_sh):
        cid = lax.axis_index("c")
        sid = lax.axis_index("s")

        def zv(i, _):
            zero_v[i] = jnp.zeros((16,), _f32)
            return 0
        lax.fori_loop(0, zb, zv, 0)

        for j in range(half):
            fc = cid * half + j
            for hh in range(2):

                @pl.when(fc < nchunk)
                def _():
                    def zslice(base, nb):
                        pltpu.sync_copy(zero_v.at[pl.ds(0, nb)],
                                        acc_sh.at[pl.ds(base, nb)])
                    row0 = sid * tzz
                    if znf:
                        def zbody(t, _):
                            zslice(row0 + t * zb, zb)
                            return 0
                        lax.fori_loop(0, znf, zbody, 0)
                    if zrem:
                        zslice(row0 + znf * zb, zrem)

                plsc.subcore_barrier()

                @pl.when(fc < nchunk)
                def _():
                    col = fc * 16

                    def scat(base, nb):
                        pltpu.sync_copy(idx_hbm.at[pl.ds(hh * m_tot + base, nb)],
                                        idx_v.at[pl.ds(0, nb)])
                        pltpu.sync_copy(
                            vals_hbm.at[pl.ds(base, nb), pl.ds(col, 16)],
                            val_v.at[pl.ds(0, nb)])
                        pltpu.sync_copy(val_v.at[pl.ds(0, nb)],
                                        acc_sh.at[idx_v.at[pl.ds(0, nb)]],
                                        add=True)

                    base0 = sid * r_ps
                    if nf:
                        def body(t, _):
                            scat(base0 + t * eb, eb)
                            return 0
                        lax.fori_loop(0, nf, body, 0)
                    if rem:
                        scat(base0 + nf * eb, rem)

                plsc.subcore_barrier()

                @pl.when(fc < nchunk)
                def _():
                    col = fc * 16
                    row0 = sid * tz
                    pltpu.sync_copy(
                        acc_sh.at[pl.ds(row0, tz)],
                        out_hbm.at[pl.ds(hh * t2 + row0, tz), pl.ds(col, 16)])

                plsc.subcore_barrier()

    return k(vals, idx2)


def _rows_pc(body, nrows, out_shapes, ins, in_widths):
    grid = (nrows // BLK,)
    in_specs = []
    for a, w in zip(ins, in_widths):
        if w is None:
            nd = a.ndim
            in_specs.append(pl.BlockSpec(a.shape, lambda i, _n=nd: (0,) * _n))
        else:
            in_specs.append(pl.BlockSpec((BLK, w), lambda i: (i, 0)))
    out_specs = [pl.BlockSpec((BLK, s.shape[1]), lambda i: (i, 0))
                 for s in out_shapes]
    if len(out_shapes) == 1:
        out_specs = out_specs[0]
        out_shape = out_shapes[0]
    else:
        out_shape = out_shapes
    return pl.pallas_call(
        body, grid=grid, in_specs=in_specs, out_specs=out_specs,
        out_shape=out_shape)(*ins)


def _k_proj(x_ref, w_ref, as_ref, ad_ref, p_ref, st_ref):
    x = x_ref[...]
    proj = jnp.dot(x, w_ref[...], preferred_element_type=_f32)
    ss = jnp.dot(proj, as_ref[...], preferred_element_type=_f32)
    p_ref[...] = jnp.concatenate(
        [proj, ss, jnp.zeros((BLK, 32), _f32)], axis=1)
    st_ref[...] = jnp.dot(proj, ad_ref[...], preferred_element_type=_f32)


def _k_scale(nrows_valid, g_ref, stdst_ref, eh_ref, sv_ref):
    i = pl.program_id(0)
    g = g_ref[...]
    s = g[:, DP:DP + 16] + stdst_ref[...]
    ex = jnp.exp(jnp.maximum(s, 0.2 * s))
    rows = i * BLK + lax.broadcasted_iota(jnp.int32, (BLK, 1), 0)
    lanes = lax.broadcasted_iota(jnp.int32, (BLK, 16), 1)
    ex = jnp.where((rows < nrows_valid) & (lanes < NH), ex, 0.0)
    scaled = g[:, :DP] * jnp.dot(ex, eh_ref[...], preferred_element_type=_f32)
    sv_ref[...] = jnp.concatenate([scaled, ex], axis=1)


def _k_epi(is_last, nrows_valid, acc_ref, x_ref, b_ref, eh_ref, *o_refs):
    acc = acc_ref[...]
    d = jnp.dot(acc[:, DP:], eh_ref[...], preferred_element_type=_f32)
    out = acc[:, :DP] / (d + 1e-16) + x_ref[..., :DP] + b_ref[...]
    if not is_last:
        o_refs[0][...] = jnp.where(
            out > 0, out, jnp.exp(jnp.minimum(out, 0.0)) - 1.0)
    else:
        i = pl.program_id(0)
        rows = i * BLK + lax.broadcasted_iota(jnp.int32, (BLK, 1), 0)
        cols = lax.broadcasted_iota(jnp.int32, (BLK, DP), 1)
        out = jnp.where(cols == D, 1.0, out)
        out = jnp.where(rows < nrows_valid, out, 0.0)
        o_refs[0][...] = out
        o_refs[1][...] = jnp.concatenate(
            [out, jnp.zeros((BLK, 48), _f32)], axis=1)


def _k_idx2(i_ref, lo_ref, hi_ref):
    ix = i_ref[...]
    tr = T2 + (ix & (TRASH - 1))
    lo_ref[...] = jnp.where(ix < T2, ix, tr)
    hi_ref[...] = jnp.where(ix >= T2, ix - T2, tr)


def _mk_idx2(idxp):
    m = idxp.shape[0]
    lo, hi = _rows_pc(
        _k_idx2, m,
        [jax.ShapeDtypeStruct((m, 1), jnp.int32),
         jax.ShapeDtypeStruct((m, 1), jnp.int32)],
        [idxp.reshape(m, 1)], [1])
    return jnp.concatenate([lo.reshape(-1), hi.reshape(-1)])


def _k_out2norm(acc_ref, o_ref):
    acc = acc_ref[...]
    cnt = jnp.maximum(acc[:, D:D + 1], 1.0)
    cols = lax.broadcasted_iota(jnp.int32, (BLK, DP), 1)
    o_ref[...] = jnp.concatenate(
        [jnp.where(cols < D, acc / cnt, 0.0), jnp.zeros((BLK, 48), _f32)],
        axis=1)


def _k_wvals(ee_ref, z_ref, lwa_ref, lwb_ref, lb_ref, wv_ref):
    ee = ee_ref[..., :DP]
    z = z_ref[..., :DP]
    ne = (jnp.sum(ee * lwa_ref[...], axis=-1, keepdims=True)
          + jnp.sum(z * lwb_ref[...], axis=-1, keepdims=True) + lb_ref[...])
    ex2 = jnp.exp(ne)
    i = pl.program_id(0)
    rows = i * BLK + lax.broadcasted_iota(jnp.int32, (BLK, 1), 0)
    ex2 = jnp.where(rows < N_SUB, ex2, 0.0)
    wv_ref[...] = jnp.concatenate([ee * ex2, z * ex2], axis=1)


def _k_subrel(ee_ref, z_ref, wa_ref, wb_ref, b_ref, o_ref):
    srl = (jnp.dot(ee_ref[..., :DP], wa_ref[...], preferred_element_type=_f32)
           + jnp.dot(z_ref[..., :DP], wb_ref[...], preferred_element_type=_f32)
           + b_ref[...])
    o_ref[...] = jnp.concatenate([srl, jnp.zeros((BLK, 48), _f32)], axis=1)


def _k_head(acc_ref, w_ref, b_ref, ent_ref, o_ref):
    acc = acc_ref[...]
    srow = 1.0 / (acc[:, D:D + 1] + 1e-16)
    head = (jnp.dot(acc, w_ref[...], preferred_element_type=_f32) * srow
            + b_ref[...])
    o_ref[...] = (jnp.concatenate([head, jnp.zeros((BLK, 48), _f32)], axis=1)
                  + ent_ref[...])


def _k_ln(x_ref, g_ref, b_ref, o_ref):
    x = x_ref[...]
    mu = jnp.sum(x, axis=-1, keepdims=True) * (1.0 / D)
    v = jnp.sum(x * x, axis=-1, keepdims=True) * (1.0 / D) - mu * mu
    o_ref[...] = (x - mu) * lax.rsqrt(v + 1e-5) * g_ref[...] + b_ref[...]


def _k_loss(ee_ref, sr_ref, see_ref, ssr_ref, o_ref):
    ee = ee_ref[...]
    sr = sr_ref[...]
    se = see_ref[...]
    ss = ssr_ref[...]
    laa = jax.nn.sigmoid(jnp.sum(ee * sr, axis=-1, keepdims=True))
    lbb = jax.nn.sigmoid(jnp.sum(se * ss, axis=-1, keepdims=True))
    lab = jax.nn.sigmoid(jnp.sum(ee * ss, axis=-1, keepdims=True))
    lba = jax.nn.sigmoid(jnp.sum(se * sr, axis=-1, keepdims=True))
    i = pl.program_id(0)
    rows = i * BLK + lax.broadcasted_iota(jnp.int32, (BLK, 1), 0)
    ok = rows < N_SUB
    p0 = jnp.sum(jnp.where(ok, jnp.maximum(lba - laa + 0.5, 0.0), 0.0))
    p1 = jnp.sum(jnp.where(ok, jnp.maximum(lab - lbb + 0.5, 0.0), 0.0))

    @pl.when(i == 0)
    def _():
        o_ref[...] = jnp.zeros((8, 128), _f32)

    lanes = lax.broadcasted_iota(jnp.int32, (8, 128), 1)
    rows8 = lax.broadcasted_iota(jnp.int32, (8, 128), 0)
    o_ref[...] += jnp.where(
        rows8 == 0, jnp.where(lanes == 0, p0, jnp.where(lanes == 1, p1, 0.0)),
        0.0)


def _k_count(b_ref, o_ref):
    i = pl.program_id(0)

    @pl.when(i == 0)
    def _():
        o_ref[...] = jnp.zeros((1, 16), _f32)

    oh = (b_ref[...] == lax.broadcasted_iota(jnp.int32, (BLK, 16), 1))
    o_ref[...] += jnp.sum(oh.astype(_f32), axis=0, keepdims=True)


def _k_relcon(rp_ref, m28_ref, w_ref, b2_ref, g_ref, b_ref, o_ref):
    rp = rp_ref[...]
    rows = lax.broadcasted_iota(jnp.int32, (32, 128), 0)
    low = jnp.where(rows < NUM_REL, rp, 0.0)
    tile14 = low + pltpu.roll(low, NUM_REL, 0)
    nr = tile14 * m28_ref[...] + rp
    y = jnp.dot(nr, w_ref[...], preferred_element_type=_f32) + b2_ref[...]
    mu = jnp.sum(y, axis=-1, keepdims=True) * (1.0 / D)
    v = jnp.sum(y * y, axis=-1, keepdims=True) * (1.0 / D) - mu * mu
    o_ref[...] = (y - mu) * lax.rsqrt(v + 1e-5) * g_ref[...] + b_ref[...]


def _k_relemb(id_ref, rc_ref, o_ref):
    oh = (id_ref[...] == lax.broadcasted_iota(jnp.int32, (BLK, 32), 1))
    o_ref[...] = jnp.dot(oh.astype(_f32), rc_ref[...],
                         preferred_element_type=_f32)


def _pad_rows(a, n):
    return jnp.pad(a, ((0, n - a.shape[0]),) + ((0, 0),) * (a.ndim - 1))


def _pad_idx(a, n):
    return jnp.pad(a.astype(jnp.int32), (0, n - a.shape[0]))


def _padw(w, r, c):
    return jnp.pad(w, ((0, r - w.shape[0]), (0, c - w.shape[1])))


def _gat_layer_pallas(xp, Wp, As16, Ad16, biasp, Eh, srcp, dstp, dst_i2, xw):
    P, stT = _rows_pc(
        _k_proj, N_PAD,
        [jax.ShapeDtypeStruct((N_PAD, DG), _f32),
         jax.ShapeDtypeStruct((N_PAD, 16), _f32)],
        [xp, Wp, As16, Ad16], [xw, None, None, None])
    stdst = _sc_gather(stT, dstp, 16, 2048)
    g = _sc_gather(P, srcp, DG, 384, tiled=True)
    sv = _rows_pc(
        functools.partial(_k_scale, E), E_PAD,
        [jax.ShapeDtypeStruct((E_PAD, DP + 16), _f32)],
        [g, stdst, Eh], [DG, 16, None])
    acc = _sc_scatter_add(sv, dst_i2, T2, DP + 16, 2048)
    return acc


def kernel(entity_emb, rel_param, W1, a_src1, a_dst1, bias1, W2, a_src2, a_dst2, bias2,
           layer_emb_W, layer_emb_b, rel_W, rel_b, sub_W, sub_b, out_W, out_b,
           ln_node_g, ln_node_b, ln_re_g, ln_re_b,
           b_x, edge_index, b_node_graph_index, sub, rel, shuf_index):
    entity_embp = jnp.pad(entity_emb, ((0, 0), (0, DG - D)))
    srcp = _pad_idx(edge_index[0], E_PAD)
    dstp = _pad_idx(edge_index[1], E_PAD)
    b_xp = _pad_idx(b_x, N_PAD)
    shufp = _pad_idx(shuf_index, N_PAD)
    subi = sub.astype(jnp.int32)

    hsel = jnp.repeat(jnp.arange(NH), FH)
    def _mk_a16(a):
        m = jnp.zeros((DP, 16), _f32)
        return m.at[jnp.arange(D), hsel].set(a.reshape(-1))
    As1, Ad1 = _mk_a16(a_src1), _mk_a16(a_dst1)
    As2, Ad2 = _mk_a16(a_src2), _mk_a16(a_dst2)
    Eh = jnp.zeros((16, DP), _f32).at[hsel, jnp.arange(D)].set(1.0)
    W1p = _padw(W1, DG, DP)
    W2p = _padw(W2, DP, DP)
    b1p = _padw(bias1.reshape(1, -1), 1, DP)
    b2p = _padw(bias2.reshape(1, -1), 1, DP)
    lwa = _padw(layer_emb_W[:D].reshape(1, -1), 1, DP)
    lwb = _padw(layer_emb_W[D:].reshape(1, -1), 1, DP)
    lb = layer_emb_b.reshape(1, 1)
    sWa = _padw(sub_W[:D], DP, DP)
    sWb = _padw(sub_W[D:], DP, DP)
    sbp = _padw(sub_b.reshape(1, -1), 1, DP)
    W6 = jnp.zeros((2 * DP, DP), _f32)
    W6 = W6.at[:D, :D].set(out_W[:D]).at[DP:DP + D, :D].set(out_W[D:])
    obp = _padw(out_b.reshape(1, -1), 1, DP)
    lngp = _padw(ln_node_g.reshape(1, -1), 1, DG)
    lnbp = _padw(ln_node_b.reshape(1, -1), 1, DG)
    lngr = _padw(ln_re_g.reshape(1, -1), 1, DP)
    lnbr = _padw(ln_re_b.reshape(1, -1), 1, DP)
    rWp = _padw(rel_W, 100, DP)
    rb2 = _padw((2.0 * rel_b).reshape(1, -1), 1, DP)
    bngi2 = jnp.pad(b_node_graph_index.astype(jnp.int32), (0, N_PAD - N_SUB),
                    constant_values=15).reshape(N_PAD, 1)

    dst_i2 = _mk_idx2(dstp)
    bx_i2 = _mk_idx2(b_xp)

    xp = _sc_gather(entity_embp, b_xp, DG, 384, tiled=True)
    acc1 = _gat_layer_pallas(xp, W1p, As1, Ad1, b1p, Eh, srcp, dstp, dst_i2,
                             DG)
    h = _rows_pc(functools.partial(_k_epi, False, N_SUB), N_PAD,
                 [jax.ShapeDtypeStruct((N_PAD, DP), _f32)],
                 [acc1, xp, b1p, Eh], [DP + 16, DG, None, None])
    acc2 = _gat_layer_pallas(h, W2p, As2, Ad2, b2p, Eh, srcp, dstp, dst_i2,
                             DP)
    eep, eep256 = _rows_pc(functools.partial(_k_epi, True, N_SUB), N_PAD,
                           [jax.ShapeDtypeStruct((N_PAD, DP), _f32),
                            jax.ShapeDtypeStruct((N_PAD, DG), _f32)],
                           [acc2, h, b2p, Eh], [DP + 16, DP, None, None])

    accB = _sc_scatter_add(eep, bx_i2, T2, DP, 2048)
    out2 = _rows_pc(_k_out2norm, N_PAD,
                    [jax.ShapeDtypeStruct((N_PAD, DG), _f32)], [accB], [DP])
    z = _sc_gather(out2, b_xp, DG, 384, tiled=True)
    ee_bx = _sc_gather(eep256, b_xp, DG, 384, tiled=True)
    wv = _rows_pc(_k_wvals, N_PAD,
                  [jax.ShapeDtypeStruct((N_PAD, 2 * DP), _f32)],
                  [ee_bx, z, lwa, lwb, lb], [DG, DG, None, None, None])
    acc3 = _sc_scatter_add(wv, bx_i2, T2, 2 * DP, 2048)
    head = _rows_pc(_k_head, N_PAD,
                    [jax.ShapeDtypeStruct((N_PAD, DG), _f32)],
                    [acc3, W6, obp, entity_embp[:N_PAD]],
                    [2 * DP, None, None, DG])
    lnin = _pad_rows(jnp.concatenate([head[:KSEG], entity_embp[KSEG:]], 0),
                     NE_PAD)
    entity_con_p = _rows_pc(_k_ln, NE_PAD,
                            [jax.ShapeDtypeStruct((NE_PAD, DG), _f32)],
                            [lnin, lngp, lnbp], [DG, None, None])
    entity_con = entity_con_p[:NUM_ENT, :D]

    cnt = pl.pallas_call(
        _k_count, grid=(N_PAD // BLK,),
        in_specs=[pl.BlockSpec((BLK, 1), lambda i: (i, 0))],
        out_specs=pl.BlockSpec((1, 16), lambda i: (0, 0)),
        out_shape=jax.ShapeDtypeStruct((1, 16), _f32))(bngi2)
    m14 = (cnt[0, :NUM_REL] > 0).astype(_f32)
    m28 = jnp.concatenate([m14, m14, jnp.zeros((4,), _f32)]).reshape(32, 1)
    rp32 = jnp.pad(rel_param, ((0, 4), (0, 28)))
    rW128 = jnp.pad(rWp, ((0, 28), (0, 0)))
    rc32 = pl.pallas_call(
        _k_relcon, grid=(1,),
        in_specs=[pl.BlockSpec((32, 128), lambda i: (0, 0)),
                  pl.BlockSpec((32, 1), lambda i: (0, 0)),
                  pl.BlockSpec((128, DP), lambda i: (0, 0)),
                  pl.BlockSpec((1, DP), lambda i: (0, 0)),
                  pl.BlockSpec((1, DP), lambda i: (0, 0)),
                  pl.BlockSpec((1, DP), lambda i: (0, 0))],
        out_specs=pl.BlockSpec((32, DP), lambda i: (0, 0)),
        out_shape=jax.ShapeDtypeStruct((32, DP), _f32))(
            rp32, m28, rW128, rb2, lngr, lnbr)
    rel_con = rc32[:2 * NUM_REL, :D]
    rel_emb = _rows_pc(_k_relemb, BQ,
                       [jax.ShapeDtypeStruct((BQ, DP), _f32)],
                       [rel.astype(jnp.int32).reshape(BQ, 1), rc32],
                       [1, None])[:, :D]

    srl = _rows_pc(_k_subrel, N_PAD,
                   [jax.ShapeDtypeStruct((N_PAD, DG), _f32)],
                   [ee_bx, z, sWa, sWb, sbp], [DG, DG, None, None, None])
    se_ = _sc_gather(eep256, shufp, DG, 384, tiled=True)
    sr_ = _sc_gather(srl, shufp, DG, 384, tiled=True)
    parts = pl.pallas_call(
        _k_loss, grid=(N_PAD // BLK,),
        in_specs=[pl.BlockSpec((BLK, DG), lambda i: (i, 0))] * 4,
        out_specs=pl.BlockSpec((8, 128), lambda i: (0, 0)),
        out_shape=jax.ShapeDtypeStruct((8, 128), _f32))(
            eep256, srl, se_, sr_)
    cl_loss = (parts[0, 0] + parts[0, 1]) / N_SUB

    sub_emb = _sc_gather(entity_con_p, subi, DG, 384, tiled=True)[:, :D]
    return (sub_emb, rel_emb, entity_con, cl_loss, rel_con)


# speedup vs baseline: 11.5487x; 1.0980x over previous
import functools

import jax
import jax.numpy as jnp
from jax import lax
from jax.experimental import pallas as pl
from jax.experimental.pallas import tpu as pltpu
from jax.experimental.pallas import tpu_sc as plsc

NUM_ENT = 72000
KSEG = 70108
NUM_REL = 14
N_SUB = 70108
E = 400000
BQ = 4096
NH = 4
FH = 50
D = 200

DP = 208
DG = 256
N_PAD = 70656
E_PAD = 400384
NE_PAD = 72192
BLK = 512

_f32 = jnp.float32


def _sc_mesh():
    return plsc.VectorSubcoreMesh(core_axis_name="c", subcore_axis_name="s",
                                  num_cores=2, num_subcores=16)


@functools.partial(jax.jit, static_argnames=("dp", "rb", "tiled"))
def _sc_gather(table, idx, dp, rb, tiled=False):
    m_tot = idx.shape[0]
    r_pw = m_tot // 32
    nf, rem = divmod(r_pw, rb)

    @functools.partial(
        pl.kernel,
        mesh=_sc_mesh(),
        out_type=jax.ShapeDtypeStruct((m_tot, dp), _f32),
        compiler_params=pltpu.CompilerParams(use_tc_tiling_on_sc=tiled),
        scratch_types=[
            pltpu.VMEM((rb,), jnp.int32),
            pltpu.VMEM((rb, dp), _f32),
            pltpu.SemaphoreType.DMA,
        ],
    )
    def k(table_hbm, idx_hbm, out_hbm, idx_v, rows_v, sem):
        wid = lax.axis_index("s") * 2 + lax.axis_index("c")
        base0 = wid * r_pw

        def do(base, nb):
            pltpu.sync_copy(idx_hbm.at[pl.ds(base, nb)], idx_v.at[pl.ds(0, nb)])
            pltpu.async_copy(
                table_hbm.at[idx_v.at[pl.ds(0, nb)]],
                rows_v.at[pl.ds(0, nb)], sem).wait()
            pltpu.sync_copy(rows_v.at[pl.ds(0, nb)], out_hbm.at[pl.ds(base, nb)])

        if nf:
            def body(j, _):
                do(base0 + j * rb, rb)
                return 0
            lax.fori_loop(0, nf, body, 0)
        if rem:
            do(base0 + nf * rb, rem)

    return k(table, idx)


T2 = 35328
TRASH = 128


@functools.partial(jax.jit, static_argnames=("t2", "dp", "eb"))
def _sc_scatter_add(vals, idx2, t2, dp, eb):
    m_tot = idx2.shape[0] // 2
    nchunk = dp // 16
    half = (nchunk + 1) // 2
    r_ps = m_tot // 16
    nf, rem = divmod(r_ps, eb)
    tz = t2 // 16
    tzz = (t2 + TRASH) // 16
    zb = min(tzz, 2048)
    znf, zrem = divmod(tzz, zb)

    @functools.partial(
        pl.kernel,
        mesh=_sc_mesh(),
        out_type=jax.ShapeDtypeStruct((2 * t2, dp), _f32),
        compiler_params=pltpu.CompilerParams(use_tc_tiling_on_sc=False),
        scratch_types=[
            pltpu.VMEM((eb,), jnp.int32),
            pltpu.VMEM((eb, 16), _f32),
            pltpu.VMEM((zb, 16), _f32),
            pltpu.VMEM_SHARED((t2 + TRASH, 16), _f32),
        ],
    )
    def k(vals_hbm, idx_hbm, out_hbm, idx_v, val_v, zero_v, acc_sh):
        cid = lax.axis_index("c")
        sid = lax.axis_index("s")

        def zv(i, _):
            zero_v[i] = jnp.zeros((16,), _f32)
            return 0
        lax.fori_loop(0, zb, zv, 0)

        for j in range(half):
            fc = cid * half + j
            for hh in range(2):

                @pl.when(fc < nchunk)
                def _():
                    def zslice(base, nb):
                        pltpu.sync_copy(zero_v.at[pl.ds(0, nb)],
                                        acc_sh.at[pl.ds(base, nb)])
                    row0 = sid * tzz
                    if znf:
                        def zbody(t, _):
                            zslice(row0 + t * zb, zb)
                            return 0
                        lax.fori_loop(0, znf, zbody, 0)
                    if zrem:
                        zslice(row0 + znf * zb, zrem)

                plsc.subcore_barrier()

                @pl.when(fc < nchunk)
                def _():
                    col = fc * 16

                    def scat(base, nb):
                        pltpu.sync_copy(idx_hbm.at[pl.ds(hh * m_tot + base, nb)],
                                        idx_v.at[pl.ds(0, nb)])
                        pltpu.sync_copy(
                            vals_hbm.at[pl.ds(base, nb), pl.ds(col, 16)],
                            val_v.at[pl.ds(0, nb)])
                        pltpu.sync_copy(val_v.at[pl.ds(0, nb)],
                                        acc_sh.at[idx_v.at[pl.ds(0, nb)]],
                                        add=True)

                    base0 = sid * r_ps
                    if nf:
                        def body(t, _):
                            scat(base0 + t * eb, eb)
                            return 0
                        lax.fori_loop(0, nf, body, 0)
                    if rem:
                        scat(base0 + nf * eb, rem)

                plsc.subcore_barrier()

                @pl.when(fc < nchunk)
                def _():
                    col = fc * 16
                    row0 = sid * tz
                    pltpu.sync_copy(
                        acc_sh.at[pl.ds(row0, tz)],
                        out_hbm.at[pl.ds(hh * t2 + row0, tz), pl.ds(col, 16)])

                plsc.subcore_barrier()

    return k(vals, idx2)


def _rows_pc(body, nrows, out_shapes, ins, in_widths):
    grid = (nrows // BLK,)
    in_specs = []
    for a, w in zip(ins, in_widths):
        if w is None:
            nd = a.ndim
            in_specs.append(pl.BlockSpec(a.shape, lambda i, _n=nd: (0,) * _n))
        else:
            in_specs.append(pl.BlockSpec((BLK, w), lambda i: (i, 0)))
    out_specs = [pl.BlockSpec((BLK, s.shape[1]), lambda i: (i, 0))
                 for s in out_shapes]
    if len(out_shapes) == 1:
        out_specs = out_specs[0]
        out_shape = out_shapes[0]
    else:
        out_shape = out_shapes
    return pl.pallas_call(
        body, grid=grid, in_specs=in_specs, out_specs=out_specs,
        out_shape=out_shape)(*ins)


def _k_proj(x_ref, w_ref, as_ref, ad_ref, p_ref, st_ref):
    x = x_ref[...]
    proj = jnp.dot(x, w_ref[...], preferred_element_type=_f32)
    ss = jnp.dot(proj, as_ref[...], preferred_element_type=_f32)
    p_ref[...] = jnp.concatenate(
        [proj, ss, jnp.zeros((BLK, 32), _f32)], axis=1)
    st_ref[...] = jnp.dot(proj, ad_ref[...], preferred_element_type=_f32)


def _k_scale(nrows_valid, g_ref, stdst_ref, eh_ref, sv_ref):
    i = pl.program_id(0)
    g = g_ref[...]
    s = g[:, DP:DP + 16] + stdst_ref[...]
    ex = jnp.exp(jnp.maximum(s, 0.2 * s))
    rows = i * BLK + lax.broadcasted_iota(jnp.int32, (BLK, 1), 0)
    lanes = lax.broadcasted_iota(jnp.int32, (BLK, 16), 1)
    ex = jnp.where((rows < nrows_valid) & (lanes < NH), ex, 0.0)
    scaled = g[:, :DP] * jnp.dot(ex, eh_ref[...], preferred_element_type=_f32)
    sv_ref[...] = jnp.concatenate([scaled, ex], axis=1)


def _k_epi(is_last, nrows_valid, acc_ref, x_ref, b_ref, eh_ref, *o_refs):
    acc = acc_ref[...]
    d = jnp.dot(acc[:, DP:], eh_ref[...], preferred_element_type=_f32)
    out = acc[:, :DP] / (d + 1e-16) + x_ref[..., :DP] + b_ref[...]
    if not is_last:
        o_refs[0][...] = jnp.where(
            out > 0, out, jnp.exp(jnp.minimum(out, 0.0)) - 1.0)
    else:
        i = pl.program_id(0)
        rows = i * BLK + lax.broadcasted_iota(jnp.int32, (BLK, 1), 0)
        cols = lax.broadcasted_iota(jnp.int32, (BLK, DP), 1)
        out = jnp.where(cols == D, 1.0, out)
        out = jnp.where(rows < nrows_valid, out, 0.0)
        o_refs[0][...] = out
        o_refs[1][...] = jnp.concatenate(
            [out, jnp.zeros((BLK, 48), _f32)], axis=1)


def _k_idx2(i_ref, lo_ref, hi_ref):
    ix = i_ref[...]
    tr = T2 + (ix & (TRASH - 1))
    lo_ref[...] = jnp.where(ix < T2, ix, tr)
    hi_ref[...] = jnp.where(ix >= T2, ix - T2, tr)


def _mk_idx2(idxp):
    m = idxp.shape[0]
    r = m // 128
    shp = jax.ShapeDtypeStruct((r, 128), jnp.int32)
    lo, hi = pl.pallas_call(
        _k_idx2, grid=(1,),
        in_specs=[pl.BlockSpec((r, 128), lambda i: (0, 0))],
        out_specs=[pl.BlockSpec((r, 128), lambda i: (0, 0))] * 2,
        out_shape=[shp, shp])(idxp.reshape(r, 128))
    return jnp.concatenate([lo.reshape(-1), hi.reshape(-1)])


def _k_out2norm(acc_ref, o_ref):
    acc = acc_ref[...]
    cnt = jnp.maximum(acc[:, D:D + 1], 1.0)
    cols = lax.broadcasted_iota(jnp.int32, (BLK, DP), 1)
    o_ref[...] = jnp.concatenate(
        [jnp.where(cols < D, acc / cnt, 0.0), jnp.zeros((BLK, 48), _f32)],
        axis=1)


def _k_wvals(ee_ref, z_ref, lwa_ref, lwb_ref, lb_ref, wv_ref):
    ee = ee_ref[..., :DP]
    z = z_ref[..., :DP]
    ne = (jnp.sum(ee * lwa_ref[...], axis=-1, keepdims=True)
          + jnp.sum(z * lwb_ref[...], axis=-1, keepdims=True) + lb_ref[...])
    ex2 = jnp.exp(ne)
    i = pl.program_id(0)
    rows = i * BLK + lax.broadcasted_iota(jnp.int32, (BLK, 1), 0)
    ex2 = jnp.where(rows < N_SUB, ex2, 0.0)
    wv_ref[...] = jnp.concatenate([ee * ex2, z * ex2], axis=1)


def _k_subrel(ee_ref, z_ref, wa_ref, wb_ref, b_ref, o_ref):
    srl = (jnp.dot(ee_ref[..., :DP], wa_ref[...], preferred_element_type=_f32)
           + jnp.dot(z_ref[..., :DP], wb_ref[...], preferred_element_type=_f32)
           + b_ref[...])
    o_ref[...] = jnp.concatenate([srl, jnp.zeros((BLK, 48), _f32)], axis=1)


def _k_head(acc_ref, w_ref, b_ref, ent_ref, o_ref):
    acc = acc_ref[...]
    srow = 1.0 / (acc[:, D:D + 1] + 1e-16)
    head = (jnp.dot(acc, w_ref[...], preferred_element_type=_f32) * srow
            + b_ref[...])
    o_ref[...] = (jnp.concatenate([head, jnp.zeros((BLK, 48), _f32)], axis=1)
                  + ent_ref[...])


def _k_ln(x_ref, g_ref, b_ref, o_ref, oc_ref):
    x = x_ref[...]
    mu = jnp.sum(x, axis=-1, keepdims=True) * (1.0 / D)
    v = jnp.sum(x * x, axis=-1, keepdims=True) * (1.0 / D) - mu * mu
    y = (x - mu) * lax.rsqrt(v + 1e-5) * g_ref[...] + b_ref[...]
    o_ref[...] = y
    oc_ref[...] = y[:, :D]


def _k_padent(x_ref, o_ref):
    o_ref[...] = jnp.concatenate(
        [x_ref[...], jnp.zeros((600, DG - D), _f32)], axis=1)


def _k_loss(ee_ref, sr_ref, see_ref, ssr_ref, o_ref):
    ee = ee_ref[...]
    sr = sr_ref[...]
    se = see_ref[...]
    ss = ssr_ref[...]
    laa = jax.nn.sigmoid(jnp.sum(ee * sr, axis=-1, keepdims=True))
    lbb = jax.nn.sigmoid(jnp.sum(se * ss, axis=-1, keepdims=True))
    lab = jax.nn.sigmoid(jnp.sum(ee * ss, axis=-1, keepdims=True))
    lba = jax.nn.sigmoid(jnp.sum(se * sr, axis=-1, keepdims=True))
    i = pl.program_id(0)
    rows = i * BLK + lax.broadcasted_iota(jnp.int32, (BLK, 1), 0)
    ok = rows < N_SUB
    p0 = jnp.sum(jnp.where(ok, jnp.maximum(lba - laa + 0.5, 0.0), 0.0))
    p1 = jnp.sum(jnp.where(ok, jnp.maximum(lab - lbb + 0.5, 0.0), 0.0))

    @pl.when(i == 0)
    def _():
        o_ref[...] = jnp.zeros((8, 128), _f32)

    lanes = lax.broadcasted_iota(jnp.int32, (8, 128), 1)
    rows8 = lax.broadcasted_iota(jnp.int32, (8, 128), 0)
    o_ref[...] += jnp.where(
        rows8 == 0, jnp.where(lanes == 0, p0, jnp.where(lanes == 1, p1, 0.0)),
        0.0)


def _k_count(b_ref, o_ref):
    blk = b_ref[...]
    lanes = lax.broadcasted_iota(jnp.int32, (1, 16), 1)
    acc = jnp.zeros((1, 16), _f32)
    for g in range(NUM_REL):
        cg = jnp.sum((blk == g).astype(_f32))
        acc = jnp.where(lanes == g, cg, acc)
    o_ref[...] = acc


def _k_relcon(rp_ref, m28_ref, w_ref, b2_ref, g_ref, b_ref, o_ref):
    rp = rp_ref[...]
    rows = lax.broadcasted_iota(jnp.int32, (32, 128), 0)
    low = jnp.where(rows < NUM_REL, rp, 0.0)
    tile14 = low + pltpu.roll(low, NUM_REL, 0)
    nr = tile14 * m28_ref[...] + rp
    y = jnp.dot(nr, w_ref[...], preferred_element_type=_f32) + b2_ref[...]
    mu = jnp.sum(y, axis=-1, keepdims=True) * (1.0 / D)
    v = jnp.sum(y * y, axis=-1, keepdims=True) * (1.0 / D) - mu * mu
    o_ref[...] = (y - mu) * lax.rsqrt(v + 1e-5) * g_ref[...] + b_ref[...]


def _k_relemb(id_ref, rc_ref, o_ref):
    oh = (id_ref[...] == lax.broadcasted_iota(jnp.int32, (BLK, 32), 1))
    o_ref[...] = jnp.dot(oh.astype(_f32), rc_ref[...],
                         preferred_element_type=_f32)


def _pad_rows(a, n):
    return jnp.pad(a, ((0, n - a.shape[0]),) + ((0, 0),) * (a.ndim - 1))


def _pad_idx(a, n):
    return jnp.pad(a.astype(jnp.int32), (0, n - a.shape[0]))


def _padw(w, r, c):
    return jnp.pad(w, ((0, r - w.shape[0]), (0, c - w.shape[1])))


def _gat_layer_pallas(xp, Wp, As16, Ad16, biasp, Eh, srcp, dstp, dst_i2, xw):
    P, stT = _rows_pc(
        _k_proj, N_PAD,
        [jax.ShapeDtypeStruct((N_PAD, DG), _f32),
         jax.ShapeDtypeStruct((N_PAD, 16), _f32)],
        [xp, Wp, As16, Ad16], [xw, None, None, None])
    stdst = _sc_gather(stT, dstp, 16, 2048)
    g = _sc_gather(P, srcp, DG, 384, tiled=True)
    sv = _rows_pc(
        functools.partial(_k_scale, E), E_PAD,
        [jax.ShapeDtypeStruct((E_PAD, DP + 16), _f32)],
        [g, stdst, Eh], [DG, 16, None])
    acc = _sc_scatter_add(sv, dst_i2, T2, DP + 16, 2048)
    return acc


def kernel(entity_emb, rel_param, W1, a_src1, a_dst1, bias1, W2, a_src2, a_dst2, bias2,
           layer_emb_W, layer_emb_b, rel_W, rel_b, sub_W, sub_b, out_W, out_b,
           ln_node_g, ln_node_b, ln_re_g, ln_re_b,
           b_x, edge_index, b_node_graph_index, sub, rel, shuf_index):
    entity_embp = pl.pallas_call(
        _k_padent, grid=(NUM_ENT // 600,),
        in_specs=[pl.BlockSpec((600, D), lambda i: (i, 0))],
        out_specs=pl.BlockSpec((600, DG), lambda i: (i, 0)),
        out_shape=jax.ShapeDtypeStruct((NUM_ENT, DG), _f32))(entity_emb)
    srcp = _pad_idx(edge_index[0], E_PAD)
    dstp = _pad_idx(edge_index[1], E_PAD)
    b_xp = _pad_idx(b_x, N_PAD)
    shufp = _pad_idx(shuf_index, N_PAD)
    subi = sub.astype(jnp.int32)

    hsel = jnp.repeat(jnp.arange(NH), FH)
    def _mk_a16(a):
        m = jnp.zeros((DP, 16), _f32)
        return m.at[jnp.arange(D), hsel].set(a.reshape(-1))
    As1, Ad1 = _mk_a16(a_src1), _mk_a16(a_dst1)
    As2, Ad2 = _mk_a16(a_src2), _mk_a16(a_dst2)
    Eh = jnp.zeros((16, DP), _f32).at[hsel, jnp.arange(D)].set(1.0)
    W1p = _padw(W1, DG, DP)
    W2p = _padw(W2, DP, DP)
    b1p = _padw(bias1.reshape(1, -1), 1, DP)
    b2p = _padw(bias2.reshape(1, -1), 1, DP)
    lwa = _padw(layer_emb_W[:D].reshape(1, -1), 1, DP)
    lwb = _padw(layer_emb_W[D:].reshape(1, -1), 1, DP)
    lb = layer_emb_b.reshape(1, 1)
    sWa = _padw(sub_W[:D], DP, DP)
    sWb = _padw(sub_W[D:], DP, DP)
    sbp = _padw(sub_b.reshape(1, -1), 1, DP)
    W6 = jnp.zeros((2 * DP, DP), _f32)
    W6 = W6.at[:D, :D].set(out_W[:D]).at[DP:DP + D, :D].set(out_W[D:])
    obp = _padw(out_b.reshape(1, -1), 1, DP)
    lngp = _padw(ln_node_g.reshape(1, -1), 1, DG)
    lnbp = _padw(ln_node_b.reshape(1, -1), 1, DG)
    lngr = _padw(ln_re_g.reshape(1, -1), 1, DP)
    lnbr = _padw(ln_re_b.reshape(1, -1), 1, DP)
    rWp = _padw(rel_W, 100, DP)
    rb2 = _padw((2.0 * rel_b).reshape(1, -1), 1, DP)
    bngi2 = jnp.pad(b_node_graph_index.astype(jnp.int32), (0, N_PAD - N_SUB),
                    constant_values=15).reshape(N_PAD // 128, 128)

    dst_i2 = _mk_idx2(dstp)
    bx_i2 = _mk_idx2(b_xp)

    xp = _sc_gather(entity_embp, b_xp, DG, 384, tiled=True)
    acc1 = _gat_layer_pallas(xp, W1p, As1, Ad1, b1p, Eh, srcp, dstp, dst_i2,
                             DG)
    h = _rows_pc(functools.partial(_k_epi, False, N_SUB), N_PAD,
                 [jax.ShapeDtypeStruct((N_PAD, DP), _f32)],
                 [acc1, xp, b1p, Eh], [DP + 16, DG, None, None])
    acc2 = _gat_layer_pallas(h, W2p, As2, Ad2, b2p, Eh, srcp, dstp, dst_i2,
                             DP)
    eep, eep256 = _rows_pc(functools.partial(_k_epi, True, N_SUB), N_PAD,
                           [jax.ShapeDtypeStruct((N_PAD, DP), _f32),
                            jax.ShapeDtypeStruct((N_PAD, DG), _f32)],
                           [acc2, h, b2p, Eh], [DP + 16, DP, None, None])

    accB = _sc_scatter_add(eep, bx_i2, T2, DP, 2048)
    out2 = _rows_pc(_k_out2norm, N_PAD,
                    [jax.ShapeDtypeStruct((N_PAD, DG), _f32)], [accB], [DP])
    z = _sc_gather(out2, b_xp, DG, 384, tiled=True)
    ee_bx = _sc_gather(eep256, b_xp, DG, 384, tiled=True)
    wv = _rows_pc(_k_wvals, N_PAD,
                  [jax.ShapeDtypeStruct((N_PAD, 2 * DP), _f32)],
                  [ee_bx, z, lwa, lwb, lb], [DG, DG, None, None, None])
    acc3 = _sc_scatter_add(wv, bx_i2, T2, 2 * DP, 2048)
    head = _rows_pc(_k_head, N_PAD,
                    [jax.ShapeDtypeStruct((N_PAD, DG), _f32)],
                    [acc3, W6, obp, entity_embp[:N_PAD]],
                    [2 * DP, None, None, DG])
    lnin = jnp.concatenate([head[:KSEG], entity_embp[KSEG:]], 0)
    entity_con_p, entity_con = pl.pallas_call(
        _k_ln, grid=(NUM_ENT // 600,),
        in_specs=[pl.BlockSpec((600, DG), lambda i: (i, 0)),
                  pl.BlockSpec((1, DG), lambda i: (0, 0)),
                  pl.BlockSpec((1, DG), lambda i: (0, 0))],
        out_specs=[pl.BlockSpec((600, DG), lambda i: (i, 0)),
                   pl.BlockSpec((600, D), lambda i: (i, 0))],
        out_shape=[jax.ShapeDtypeStruct((NUM_ENT, DG), _f32),
                   jax.ShapeDtypeStruct((NUM_ENT, D), _f32)])(
            lnin, lngp, lnbp)

    cnt = pl.pallas_call(
        _k_count, grid=(1,),
        in_specs=[pl.BlockSpec((N_PAD // 128, 128), lambda i: (0, 0))],
        out_specs=pl.BlockSpec((1, 16), lambda i: (0, 0)),
        out_shape=jax.ShapeDtypeStruct((1, 16), _f32))(bngi2)
    m14 = (cnt[0, :NUM_REL] > 0).astype(_f32)
    m28 = jnp.concatenate([m14, m14, jnp.zeros((4,), _f32)]).reshape(32, 1)
    rp32 = jnp.pad(rel_param, ((0, 4), (0, 28)))
    rW128 = jnp.pad(rWp, ((0, 28), (0, 0)))
    rc32 = pl.pallas_call(
        _k_relcon, grid=(1,),
        in_specs=[pl.BlockSpec((32, 128), lambda i: (0, 0)),
                  pl.BlockSpec((32, 1), lambda i: (0, 0)),
                  pl.BlockSpec((128, DP), lambda i: (0, 0)),
                  pl.BlockSpec((1, DP), lambda i: (0, 0)),
                  pl.BlockSpec((1, DP), lambda i: (0, 0)),
                  pl.BlockSpec((1, DP), lambda i: (0, 0))],
        out_specs=pl.BlockSpec((32, DP), lambda i: (0, 0)),
        out_shape=jax.ShapeDtypeStruct((32, DP), _f32))(
            rp32, m28, rW128, rb2, lngr, lnbr)
    rel_con = rc32[:2 * NUM_REL, :D]
    rel_emb = _rows_pc(_k_relemb, BQ,
                       [jax.ShapeDtypeStruct((BQ, DP), _f32)],
                       [rel.astype(jnp.int32).reshape(BQ, 1), rc32],
                       [1, None])[:, :D]

    srl = _rows_pc(_k_subrel, N_PAD,
                   [jax.ShapeDtypeStruct((N_PAD, DG), _f32)],
                   [ee_bx, z, sWa, sWb, sbp], [DG, DG, None, None, None])
    se_ = _sc_gather(eep256, shufp, DG, 384, tiled=True)
    sr_ = _sc_gather(srl, shufp, DG, 384, tiled=True)
    parts = pl.pallas_call(
        _k_loss, grid=(N_PAD // BLK,),
        in_specs=[pl.BlockSpec((BLK, DG), lambda i: (i, 0))] * 4,
        out_specs=pl.BlockSpec((8, 128), lambda i: (0, 0)),
        out_shape=jax.ShapeDtypeStruct((8, 128), _f32))(
            eep256, srl, se_, sr_)
    cl_loss = (parts[0, 0] + parts[0, 1]) / N_SUB

    sub_emb = _sc_gather(entity_con_p, subi, DG, 384, tiled=True)[:, :D]
    return (sub_emb, rel_emb, entity_con, cl_loss, rel_con)


# speedup vs baseline: 11.8488x; 1.0260x over previous
import functools

import jax
import jax.numpy as jnp
from jax import lax
from jax.experimental import pallas as pl
from jax.experimental.pallas import tpu as pltpu
from jax.experimental.pallas import tpu_sc as plsc

NUM_ENT = 72000
KSEG = 70108
NUM_REL = 14
N_SUB = 70108
E = 400000
BQ = 4096
NH = 4
FH = 50
D = 200

DP = 208
DG = 256
N_PAD = 70656
E_PAD = 400384
NE_PAD = 72192
BLK = 512

_f32 = jnp.float32


def _sc_mesh():
    return plsc.VectorSubcoreMesh(core_axis_name="c", subcore_axis_name="s",
                                  num_cores=2, num_subcores=16)


@functools.partial(jax.jit, static_argnames=("dp", "rb", "tiled"))
def _sc_gather(table, idx, dp, rb, tiled=False):
    m_tot = idx.shape[0]
    r_pw = m_tot // 32
    nf, rem = divmod(r_pw, rb)

    @functools.partial(
        pl.kernel,
        mesh=_sc_mesh(),
        out_type=jax.ShapeDtypeStruct((m_tot, dp), _f32),
        compiler_params=pltpu.CompilerParams(use_tc_tiling_on_sc=tiled),
        scratch_types=[
            pltpu.VMEM((rb,), jnp.int32),
            pltpu.VMEM((rb, dp), _f32),
            pltpu.SemaphoreType.DMA,
        ],
    )
    def k(table_hbm, idx_hbm, out_hbm, idx_v, rows_v, sem):
        wid = lax.axis_index("s") * 2 + lax.axis_index("c")
        base0 = wid * r_pw

        def do(base, nb):
            pltpu.sync_copy(idx_hbm.at[pl.ds(base, nb)], idx_v.at[pl.ds(0, nb)])
            pltpu.async_copy(
                table_hbm.at[idx_v.at[pl.ds(0, nb)]],
                rows_v.at[pl.ds(0, nb)], sem).wait()
            pltpu.sync_copy(rows_v.at[pl.ds(0, nb)], out_hbm.at[pl.ds(base, nb)])

        if nf:
            def body(j, _):
                do(base0 + j * rb, rb)
                return 0
            lax.fori_loop(0, nf, body, 0)
        if rem:
            do(base0 + nf * rb, rem)

    return k(table, idx)


T2 = 35328
TRASH = 128


@functools.partial(jax.jit, static_argnames=("t2", "dp", "eb"))
def _sc_scatter_add(vals, idx2, t2, dp, eb):
    m_tot = idx2.shape[0] // 2
    nchunk = dp // 16
    half = (nchunk + 1) // 2
    r_ps = m_tot // 16
    nf, rem = divmod(r_ps, eb)
    tz = t2 // 16
    tzz = (t2 + TRASH) // 16
    zb = min(tzz, 2048)
    znf, zrem = divmod(tzz, zb)

    npair = nf // 2

    @functools.partial(
        pl.kernel,
        mesh=_sc_mesh(),
        out_type=jax.ShapeDtypeStruct((2 * t2, dp), _f32),
        compiler_params=pltpu.CompilerParams(use_tc_tiling_on_sc=False),
        scratch_types=[
            pltpu.VMEM((r_ps,), jnp.int32),
            pltpu.VMEM((eb, 16), _f32),
            pltpu.VMEM((zb, 16), _f32),
            pltpu.VMEM_SHARED((t2 + TRASH, 16), _f32),
        ],
    )
    def k(vals_hbm, idx_hbm, out_hbm, idx_all, val_a, zero_v, acc_sh):
        cid = lax.axis_index("c")
        sid = lax.axis_index("s")
        base0 = sid * r_ps

        def zv(i, _):
            zero_v[i] = jnp.zeros((16,), _f32)
            return 0
        lax.fori_loop(0, zb, zv, 0)

        for hh in range(2):
            pltpu.sync_copy(idx_hbm.at[pl.ds(hh * m_tot + base0, r_ps)],
                            idx_all)
            for j in range(half):
                fc = cid * half + j

                @pl.when(fc < nchunk)
                def _():
                    def zslice(base, nb):
                        pltpu.sync_copy(zero_v.at[pl.ds(0, nb)],
                                        acc_sh.at[pl.ds(base, nb)])
                    row0 = sid * tzz
                    if znf:
                        def zbody(t, _):
                            zslice(row0 + t * zb, zb)
                            return 0
                        lax.fori_loop(0, znf, zbody, 0)
                    if zrem:
                        zslice(row0 + znf * zb, zrem)

                plsc.subcore_barrier()

                @pl.when(fc < nchunk)
                def _():
                    col = fc * 16

                    def scat(t, nb):
                        pltpu.sync_copy(
                            vals_hbm.at[pl.ds(base0 + t * eb, nb),
                                        pl.ds(col, 16)],
                            val_a.at[pl.ds(0, nb)])
                        pltpu.sync_copy(
                            val_a.at[pl.ds(0, nb)],
                            acc_sh.at[idx_all.at[pl.ds(t * eb, nb)]],
                            add=True)

                    if nf:
                        def body(t, _):
                            scat(t, eb)
                            return 0
                        lax.fori_loop(0, nf, body, 0)
                    if rem:
                        scat(nf, rem)

                plsc.subcore_barrier()

                @pl.when(fc < nchunk)
                def _():
                    col = fc * 16
                    row0 = sid * tz
                    pltpu.sync_copy(
                        acc_sh.at[pl.ds(row0, tz)],
                        out_hbm.at[pl.ds(hh * t2 + row0, tz), pl.ds(col, 16)])

                plsc.subcore_barrier()

    return k(vals, idx2)


def _rows_pc(body, nrows, out_shapes, ins, in_widths):
    grid = (nrows // BLK,)
    in_specs = []
    for a, w in zip(ins, in_widths):
        if w is None:
            nd = a.ndim
            in_specs.append(pl.BlockSpec(a.shape, lambda i, _n=nd: (0,) * _n))
        else:
            in_specs.append(pl.BlockSpec((BLK, w), lambda i: (i, 0)))
    out_specs = [pl.BlockSpec((BLK, s.shape[1]), lambda i: (i, 0))
                 for s in out_shapes]
    if len(out_shapes) == 1:
        out_specs = out_specs[0]
        out_shape = out_shapes[0]
    else:
        out_shape = out_shapes
    return pl.pallas_call(
        body, grid=grid, in_specs=in_specs, out_specs=out_specs,
        out_shape=out_shape)(*ins)


def _k_proj(x_ref, w_ref, as_ref, ad_ref, p_ref, st_ref):
    x = x_ref[...]
    proj = jnp.dot(x, w_ref[...], preferred_element_type=_f32)
    ss = jnp.dot(proj, as_ref[...], preferred_element_type=_f32)
    p_ref[...] = jnp.concatenate(
        [proj, ss, jnp.zeros((BLK, 32), _f32)], axis=1)
    st_ref[...] = jnp.dot(proj, ad_ref[...], preferred_element_type=_f32)


def _k_scale(nrows_valid, g_ref, stdst_ref, eh_ref, sv_ref):
    i = pl.program_id(0)
    g = g_ref[...]
    s = g[:, DP:DP + 16] + stdst_ref[...]
    ex = jnp.exp(jnp.maximum(s, 0.2 * s))
    rows = i * BLK + lax.broadcasted_iota(jnp.int32, (BLK, 1), 0)
    lanes = lax.broadcasted_iota(jnp.int32, (BLK, 16), 1)
    ex = jnp.where((rows < nrows_valid) & (lanes < NH), ex, 0.0)
    scaled = g[:, :DP] * jnp.dot(ex, eh_ref[...], preferred_element_type=_f32)
    sv_ref[...] = jnp.concatenate([scaled, ex], axis=1)


def _k_epi(is_last, nrows_valid, acc_ref, x_ref, b_ref, eh_ref, *o_refs):
    acc = acc_ref[...]
    d = jnp.dot(acc[:, DP:], eh_ref[...], preferred_element_type=_f32)
    out = acc[:, :DP] / (d + 1e-16) + x_ref[..., :DP] + b_ref[...]
    if not is_last:
        o_refs[0][...] = jnp.where(
            out > 0, out, jnp.exp(jnp.minimum(out, 0.0)) - 1.0)
    else:
        i = pl.program_id(0)
        rows = i * BLK + lax.broadcasted_iota(jnp.int32, (BLK, 1), 0)
        cols = lax.broadcasted_iota(jnp.int32, (BLK, DP), 1)
        out = jnp.where(cols == D, 1.0, out)
        out = jnp.where(rows < nrows_valid, out, 0.0)
        o_refs[0][...] = out
        o_refs[1][...] = jnp.concatenate(
            [out, jnp.zeros((BLK, 48), _f32)], axis=1)


def _k_idx2(i_ref, lo_ref, hi_ref):
    ix = i_ref[...]
    tr = T2 + (ix & (TRASH - 1))
    lo_ref[...] = jnp.where(ix < T2, ix, tr)
    hi_ref[...] = jnp.where(ix >= T2, ix - T2, tr)


def _mk_idx2(idxp):
    m = idxp.shape[0]
    r = m // 128
    shp = jax.ShapeDtypeStruct((r, 128), jnp.int32)
    lo, hi = pl.pallas_call(
        _k_idx2, grid=(1,),
        in_specs=[pl.BlockSpec((r, 128), lambda i: (0, 0))],
        out_specs=[pl.BlockSpec((r, 128), lambda i: (0, 0))] * 2,
        out_shape=[shp, shp])(idxp.reshape(r, 128))
    return jnp.concatenate([lo.reshape(-1), hi.reshape(-1)])


def _k_out2norm(acc_ref, o_ref):
    acc = acc_ref[...]
    cnt = jnp.maximum(acc[:, D:D + 1], 1.0)
    cols = lax.broadcasted_iota(jnp.int32, (BLK, DP), 1)
    o_ref[...] = jnp.concatenate(
        [jnp.where(cols < D, acc / cnt, 0.0), jnp.zeros((BLK, 48), _f32)],
        axis=1)


def _k_wvals(ee_ref, z_ref, lwa_ref, lwb_ref, lb_ref, wv_ref):
    ee = ee_ref[..., :DP]
    z = z_ref[..., :DP]
    ne = (jnp.sum(ee * lwa_ref[...], axis=-1, keepdims=True)
          + jnp.sum(z * lwb_ref[...], axis=-1, keepdims=True) + lb_ref[...])
    ex2 = jnp.exp(ne)
    i = pl.program_id(0)
    rows = i * BLK + lax.broadcasted_iota(jnp.int32, (BLK, 1), 0)
    ex2 = jnp.where(rows < N_SUB, ex2, 0.0)
    wv_ref[...] = jnp.concatenate([ee * ex2, z * ex2], axis=1)


def _k_subrel(ee_ref, z_ref, wa_ref, wb_ref, b_ref, o_ref):
    srl = (jnp.dot(ee_ref[..., :DP], wa_ref[...], preferred_element_type=_f32)
           + jnp.dot(z_ref[..., :DP], wb_ref[...], preferred_element_type=_f32)
           + b_ref[...])
    o_ref[...] = jnp.concatenate([srl, jnp.zeros((BLK, 48), _f32)], axis=1)


def _k_head(acc_ref, w_ref, b_ref, ent_ref, o_ref):
    acc = acc_ref[...]
    srow = 1.0 / (acc[:, D:D + 1] + 1e-16)
    head = (jnp.dot(acc, w_ref[...], preferred_element_type=_f32) * srow
            + b_ref[...])
    o_ref[...] = (jnp.concatenate([head, jnp.zeros((BLK, 48), _f32)], axis=1)
                  + ent_ref[...])


def _k_ln(x_ref, g_ref, b_ref, o_ref, oc_ref):
    x = x_ref[...]
    mu = jnp.sum(x, axis=-1, keepdims=True) * (1.0 / D)
    v = jnp.sum(x * x, axis=-1, keepdims=True) * (1.0 / D) - mu * mu
    y = (x - mu) * lax.rsqrt(v + 1e-5) * g_ref[...] + b_ref[...]
    o_ref[...] = y
    oc_ref[...] = y[:, :D]


def _k_padent(x_ref, o_ref):
    o_ref[...] = jnp.concatenate(
        [x_ref[...], jnp.zeros((600, DG - D), _f32)], axis=1)


def _k_loss(ee_ref, sr_ref, see_ref, ssr_ref, o_ref):
    ee = ee_ref[...]
    sr = sr_ref[...]
    se = see_ref[...]
    ss = ssr_ref[...]
    laa = jax.nn.sigmoid(jnp.sum(ee * sr, axis=-1, keepdims=True))
    lbb = jax.nn.sigmoid(jnp.sum(se * ss, axis=-1, keepdims=True))
    lab = jax.nn.sigmoid(jnp.sum(ee * ss, axis=-1, keepdims=True))
    lba = jax.nn.sigmoid(jnp.sum(se * sr, axis=-1, keepdims=True))
    i = pl.program_id(0)
    rows = i * BLK + lax.broadcasted_iota(jnp.int32, (BLK, 1), 0)
    ok = rows < N_SUB
    p0 = jnp.sum(jnp.where(ok, jnp.maximum(lba - laa + 0.5, 0.0), 0.0))
    p1 = jnp.sum(jnp.where(ok, jnp.maximum(lab - lbb + 0.5, 0.0), 0.0))

    @pl.when(i == 0)
    def _():
        o_ref[...] = jnp.zeros((8, 128), _f32)

    lanes = lax.broadcasted_iota(jnp.int32, (8, 128), 1)
    rows8 = lax.broadcasted_iota(jnp.int32, (8, 128), 0)
    o_ref[...] += jnp.where(
        rows8 == 0, jnp.where(lanes == 0, p0, jnp.where(lanes == 1, p1, 0.0)),
        0.0)


def _k_count(b_ref, o_ref):
    blk = b_ref[...]
    lanes = lax.broadcasted_iota(jnp.int32, (1, 16), 1)
    acc = jnp.zeros((1, 16), _f32)
    for g in range(NUM_REL):
        cg = jnp.sum((blk == g).astype(_f32))
        acc = jnp.where(lanes == g, cg, acc)
    o_ref[...] = acc


def _k_relcon(rp_ref, m28_ref, w_ref, b2_ref, g_ref, b_ref, o_ref):
    rp = rp_ref[...]
    rows = lax.broadcasted_iota(jnp.int32, (32, 128), 0)
    low = jnp.where(rows < NUM_REL, rp, 0.0)
    tile14 = low + pltpu.roll(low, NUM_REL, 0)
    nr = tile14 * m28_ref[...] + rp
    y = jnp.dot(nr, w_ref[...], preferred_element_type=_f32) + b2_ref[...]
    mu = jnp.sum(y, axis=-1, keepdims=True) * (1.0 / D)
    v = jnp.sum(y * y, axis=-1, keepdims=True) * (1.0 / D) - mu * mu
    o_ref[...] = (y - mu) * lax.rsqrt(v + 1e-5) * g_ref[...] + b_ref[...]


def _k_relemb(id_ref, rc_ref, o_ref):
    oh = (id_ref[...] == lax.broadcasted_iota(jnp.int32, (BLK, 32), 1))
    o_ref[...] = jnp.dot(oh.astype(_f32), rc_ref[...],
                         preferred_element_type=_f32)


def _pad_rows(a, n):
    return jnp.pad(a, ((0, n - a.shape[0]),) + ((0, 0),) * (a.ndim - 1))


def _pad_idx(a, n):
    return jnp.pad(a.astype(jnp.int32), (0, n - a.shape[0]))


def _padw(w, r, c):
    return jnp.pad(w, ((0, r - w.shape[0]), (0, c - w.shape[1])))


def _gat_layer_pallas(xp, Wp, As16, Ad16, biasp, Eh, srcp, dstp, dst_i2, xw):
    P, stT = _rows_pc(
        _k_proj, N_PAD,
        [jax.ShapeDtypeStruct((N_PAD, DG), _f32),
         jax.ShapeDtypeStruct((N_PAD, 16), _f32)],
        [xp, Wp, As16, Ad16], [xw, None, None, None])
    stdst = _sc_gather(stT, dstp, 16, 2048)
    g = _sc_gather(P, srcp, DG, 384, tiled=True)
    sv = _rows_pc(
        functools.partial(_k_scale, E), E_PAD,
        [jax.ShapeDtypeStruct((E_PAD, DP + 16), _f32)],
        [g, stdst, Eh], [DG, 16, None])
    acc = _sc_scatter_add(sv, dst_i2, T2, DP + 16, 2048)
    return acc


def kernel(entity_emb, rel_param, W1, a_src1, a_dst1, bias1, W2, a_src2, a_dst2, bias2,
           layer_emb_W, layer_emb_b, rel_W, rel_b, sub_W, sub_b, out_W, out_b,
           ln_node_g, ln_node_b, ln_re_g, ln_re_b,
           b_x, edge_index, b_node_graph_index, sub, rel, shuf_index):
    entity_embp = pl.pallas_call(
        _k_padent, grid=(NUM_ENT // 600,),
        in_specs=[pl.BlockSpec((600, D), lambda i: (i, 0))],
        out_specs=pl.BlockSpec((600, DG), lambda i: (i, 0)),
        out_shape=jax.ShapeDtypeStruct((NUM_ENT, DG), _f32))(entity_emb)
    srcp = _pad_idx(edge_index[0], E_PAD)
    dstp = _pad_idx(edge_index[1], E_PAD)
    b_xp = _pad_idx(b_x, N_PAD)
    shufp = _pad_idx(shuf_index, N_PAD)
    subi = sub.astype(jnp.int32)

    hsel = jnp.repeat(jnp.arange(NH), FH)
    def _mk_a16(a):
        m = jnp.zeros((DP, 16), _f32)
        return m.at[jnp.arange(D), hsel].set(a.reshape(-1))
    As1, Ad1 = _mk_a16(a_src1), _mk_a16(a_dst1)
    As2, Ad2 = _mk_a16(a_src2), _mk_a16(a_dst2)
    Eh = jnp.zeros((16, DP), _f32).at[hsel, jnp.arange(D)].set(1.0)
    W1p = _padw(W1, DG, DP)
    W2p = _padw(W2, DP, DP)
    b1p = _padw(bias1.reshape(1, -1), 1, DP)
    b2p = _padw(bias2.reshape(1, -1), 1, DP)
    lwa = _padw(layer_emb_W[:D].reshape(1, -1), 1, DP)
    lwb = _padw(layer_emb_W[D:].reshape(1, -1), 1, DP)
    lb = layer_emb_b.reshape(1, 1)
    sWa = _padw(sub_W[:D], DP, DP)
    sWb = _padw(sub_W[D:], DP, DP)
    sbp = _padw(sub_b.reshape(1, -1), 1, DP)
    W6 = jnp.zeros((2 * DP, DP), _f32)
    W6 = W6.at[:D, :D].set(out_W[:D]).at[DP:DP + D, :D].set(out_W[D:])
    obp = _padw(out_b.reshape(1, -1), 1, DP)
    lngp = _padw(ln_node_g.reshape(1, -1), 1, DG)
    lnbp = _padw(ln_node_b.reshape(1, -1), 1, DG)
    lngr = _padw(ln_re_g.reshape(1, -1), 1, DP)
    lnbr = _padw(ln_re_b.reshape(1, -1), 1, DP)
    rWp = _padw(rel_W, 100, DP)
    rb2 = _padw((2.0 * rel_b).reshape(1, -1), 1, DP)
    bngi2 = jnp.pad(b_node_graph_index.astype(jnp.int32), (0, N_PAD - N_SUB),
                    constant_values=15).reshape(N_PAD // 128, 128)

    dst_i2 = _mk_idx2(dstp)
    bx_i2 = _mk_idx2(b_xp)

    xp = _sc_gather(entity_embp, b_xp, DG, 384, tiled=True)
    acc1 = _gat_layer_pallas(xp, W1p, As1, Ad1, b1p, Eh, srcp, dstp, dst_i2,
                             DG)
    h = _rows_pc(functools.partial(_k_epi, False, N_SUB), N_PAD,
                 [jax.ShapeDtypeStruct((N_PAD, DP), _f32)],
                 [acc1, xp, b1p, Eh], [DP + 16, DG, None, None])
    acc2 = _gat_layer_pallas(h, W2p, As2, Ad2, b2p, Eh, srcp, dstp, dst_i2,
                             DP)
    eep, eep256 = _rows_pc(functools.partial(_k_epi, True, N_SUB), N_PAD,
                           [jax.ShapeDtypeStruct((N_PAD, DP), _f32),
                            jax.ShapeDtypeStruct((N_PAD, DG), _f32)],
                           [acc2, h, b2p, Eh], [DP + 16, DP, None, None])

    accB = _sc_scatter_add(eep, bx_i2, T2, DP, 2048)
    out2 = _rows_pc(_k_out2norm, N_PAD,
                    [jax.ShapeDtypeStruct((N_PAD, DG), _f32)], [accB], [DP])
    z = _sc_gather(out2, b_xp, DG, 384, tiled=True)
    ee_bx = _sc_gather(eep256, b_xp, DG, 384, tiled=True)
    wv = _rows_pc(_k_wvals, N_PAD,
                  [jax.ShapeDtypeStruct((N_PAD, 2 * DP), _f32)],
                  [ee_bx, z, lwa, lwb, lb], [DG, DG, None, None, None])
    acc3 = _sc_scatter_add(wv, bx_i2, T2, 2 * DP, 2048)
    head = _rows_pc(_k_head, N_PAD,
                    [jax.ShapeDtypeStruct((N_PAD, DG), _f32)],
                    [acc3, W6, obp, entity_embp[:N_PAD]],
                    [2 * DP, None, None, DG])
    lnin = jnp.concatenate([head[:KSEG], entity_embp[KSEG:]], 0)
    entity_con_p, entity_con = pl.pallas_call(
        _k_ln, grid=(NUM_ENT // 600,),
        in_specs=[pl.BlockSpec((600, DG), lambda i: (i, 0)),
                  pl.BlockSpec((1, DG), lambda i: (0, 0)),
                  pl.BlockSpec((1, DG), lambda i: (0, 0))],
        out_specs=[pl.BlockSpec((600, DG), lambda i: (i, 0)),
                   pl.BlockSpec((600, D), lambda i: (i, 0))],
        out_shape=[jax.ShapeDtypeStruct((NUM_ENT, DG), _f32),
                   jax.ShapeDtypeStruct((NUM_ENT, D), _f32)])(
            lnin, lngp, lnbp)

    cnt = pl.pallas_call(
        _k_count, grid=(1,),
        in_specs=[pl.BlockSpec((N_PAD // 128, 128), lambda i: (0, 0))],
        out_specs=pl.BlockSpec((1, 16), lambda i: (0, 0)),
        out_shape=jax.ShapeDtypeStruct((1, 16), _f32))(bngi2)
    m14 = (cnt[0, :NUM_REL] > 0).astype(_f32)
    m28 = jnp.concatenate([m14, m14, jnp.zeros((4,), _f32)]).reshape(32, 1)
    rp32 = jnp.pad(rel_param, ((0, 4), (0, 28)))
    rW128 = jnp.pad(rWp, ((0, 28), (0, 0)))
    rc32 = pl.pallas_call(
        _k_relcon, grid=(1,),
        in_specs=[pl.BlockSpec((32, 128), lambda i: (0, 0)),
                  pl.BlockSpec((32, 1), lambda i: (0, 0)),
                  pl.BlockSpec((128, DP), lambda i: (0, 0)),
                  pl.BlockSpec((1, DP), lambda i: (0, 0)),
                  pl.BlockSpec((1, DP), lambda i: (0, 0)),
                  pl.BlockSpec((1, DP), lambda i: (0, 0))],
        out_specs=pl.BlockSpec((32, DP), lambda i: (0, 0)),
        out_shape=jax.ShapeDtypeStruct((32, DP), _f32))(
            rp32, m28, rW128, rb2, lngr, lnbr)
    rel_con = rc32[:2 * NUM_REL, :D]
    rel_emb = _rows_pc(_k_relemb, BQ,
                       [jax.ShapeDtypeStruct((BQ, DP), _f32)],
                       [rel.astype(jnp.int32).reshape(BQ, 1), rc32],
                       [1, None])[:, :D]

    srl = _rows_pc(_k_subrel, N_PAD,
                   [jax.ShapeDtypeStruct((N_PAD, DG), _f32)],
                   [ee_bx, z, sWa, sWb, sbp], [DG, DG, None, None, None])
    se_ = _sc_gather(eep256, shufp, DG, 384, tiled=True)
    sr_ = _sc_gather(srl, shufp, DG, 384, tiled=True)
    parts = pl.pallas_call(
        _k_loss, grid=(N_PAD // BLK,),
        in_specs=[pl.BlockSpec((BLK, DG), lambda i: (i, 0))] * 4,
        out_specs=pl.BlockSpec((8, 128), lambda i: (0, 0)),
        out_shape=jax.ShapeDtypeStruct((8, 128), _f32))(
            eep256, srl, se_, sr_)
    cl_loss = (parts[0, 0] + parts[0, 1]) / N_SUB

    sub_emb = _sc_gather(entity_con_p, subi, DG, 384, tiled=True)[:, :D]
    return (sub_emb, rel_emb, entity_con, cl_loss, rel_con)


# speedup vs baseline: 11.8625x; 1.0012x over previous
import functools

import jax
import jax.numpy as jnp
from jax import lax
from jax.experimental import pallas as pl
from jax.experimental.pallas import tpu as pltpu
from jax.experimental.pallas import tpu_sc as plsc

NUM_ENT = 72000
KSEG = 70108
NUM_REL = 14
N_SUB = 70108
E = 400000
BQ = 4096
NH = 4
FH = 50
D = 200

DP = 208
DG = 256
N_PAD = 70656
E_PAD = 400384
NE_PAD = 72192
BLK = 512

_f32 = jnp.float32


def _sc_mesh():
    return plsc.VectorSubcoreMesh(core_axis_name="c", subcore_axis_name="s",
                                  num_cores=2, num_subcores=16)


@functools.partial(jax.jit, static_argnames=("dp", "rb", "tiled"))
def _sc_gather(table, idx, dp, rb, tiled=False):
    m_tot = idx.shape[0]
    r_pw = m_tot // 32
    nf, rem = divmod(r_pw, rb)

    @functools.partial(
        pl.kernel,
        mesh=_sc_mesh(),
        out_type=jax.ShapeDtypeStruct((m_tot, dp), _f32),
        compiler_params=pltpu.CompilerParams(use_tc_tiling_on_sc=tiled),
        scratch_types=[
            pltpu.VMEM((rb,), jnp.int32),
            pltpu.VMEM((rb, dp), _f32),
            pltpu.SemaphoreType.DMA,
        ],
    )
    def k(table_hbm, idx_hbm, out_hbm, idx_v, rows_v, sem):
        wid = lax.axis_index("s") * 2 + lax.axis_index("c")
        base0 = wid * r_pw

        def do(base, nb):
            pltpu.sync_copy(idx_hbm.at[pl.ds(base, nb)], idx_v.at[pl.ds(0, nb)])
            pltpu.async_copy(
                table_hbm.at[idx_v.at[pl.ds(0, nb)]],
                rows_v.at[pl.ds(0, nb)], sem).wait()
            pltpu.sync_copy(rows_v.at[pl.ds(0, nb)], out_hbm.at[pl.ds(base, nb)])

        if nf:
            def body(j, _):
                do(base0 + j * rb, rb)
                return 0
            lax.fori_loop(0, nf, body, 0)
        if rem:
            do(base0 + nf * rb, rem)

    return k(table, idx)


T2 = 35328
TRASH = 128


@functools.partial(jax.jit, static_argnames=("t2", "dp", "eb"))
def _sc_scatter_add(vals, idx2, t2, dp, eb):
    m_tot = idx2.shape[0] // 2
    nchunk = dp // 16
    half = (nchunk + 1) // 2
    r_ps = m_tot // 16
    nf, rem = divmod(r_ps, eb)
    tz = t2 // 16
    tzz = (t2 + TRASH) // 16
    zb = min(tzz, 1024)
    znf, zrem = divmod(tzz, zb)

    npair = nf // 2

    @functools.partial(
        pl.kernel,
        mesh=_sc_mesh(),
        out_type=jax.ShapeDtypeStruct((2 * t2, dp), _f32),
        compiler_params=pltpu.CompilerParams(use_tc_tiling_on_sc=False),
        scratch_types=[
            pltpu.VMEM((r_ps,), jnp.int32),
            pltpu.VMEM((eb, 16), _f32),
            pltpu.VMEM((zb, 16), _f32),
            pltpu.VMEM_SHARED((t2 + TRASH, 16), _f32),
        ],
    )
    def k(vals_hbm, idx_hbm, out_hbm, idx_all, val_a, zero_v, acc_sh):
        cid = lax.axis_index("c")
        sid = lax.axis_index("s")
        base0 = sid * r_ps

        def zv(i, _):
            zero_v[i] = jnp.zeros((16,), _f32)
            return 0
        lax.fori_loop(0, zb, zv, 0)

        for hh in range(2):
            pltpu.sync_copy(idx_hbm.at[pl.ds(hh * m_tot + base0, r_ps)],
                            idx_all)
            for j in range(half):
                fc = cid * half + j

                @pl.when(fc < nchunk)
                def _():
                    def zslice(base, nb):
                        pltpu.sync_copy(zero_v.at[pl.ds(0, nb)],
                                        acc_sh.at[pl.ds(base, nb)])
                    row0 = sid * tzz
                    if znf:
                        def zbody(t, _):
                            zslice(row0 + t * zb, zb)
                            return 0
                        lax.fori_loop(0, znf, zbody, 0)
                    if zrem:
                        zslice(row0 + znf * zb, zrem)

                plsc.subcore_barrier()

                @pl.when(fc < nchunk)
                def _():
                    col = fc * 16

                    def scat(t, nb):
                        pltpu.sync_copy(
                            vals_hbm.at[pl.ds(base0 + t * eb, nb),
                                        pl.ds(col, 16)],
                            val_a.at[pl.ds(0, nb)])
                        pltpu.sync_copy(
                            val_a.at[pl.ds(0, nb)],
                            acc_sh.at[idx_all.at[pl.ds(t * eb, nb)]],
                            add=True)

                    if nf:
                        def body(t, _):
                            scat(t, eb)
                            return 0
                        lax.fori_loop(0, nf, body, 0)
                    if rem:
                        scat(nf, rem)

                plsc.subcore_barrier()

                @pl.when(fc < nchunk)
                def _():
                    col = fc * 16
                    row0 = sid * tz
                    pltpu.sync_copy(
                        acc_sh.at[pl.ds(row0, tz)],
                        out_hbm.at[pl.ds(hh * t2 + row0, tz), pl.ds(col, 16)])

                plsc.subcore_barrier()

    return k(vals, idx2)


def _rows_pc(body, nrows, out_shapes, ins, in_widths):
    grid = (nrows // BLK,)
    in_specs = []
    for a, w in zip(ins, in_widths):
        if w is None:
            nd = a.ndim
            in_specs.append(pl.BlockSpec(a.shape, lambda i, _n=nd: (0,) * _n))
        else:
            in_specs.append(pl.BlockSpec((BLK, w), lambda i: (i, 0)))
    out_specs = [pl.BlockSpec((BLK, s.shape[1]), lambda i: (i, 0))
                 for s in out_shapes]
    if len(out_shapes) == 1:
        out_specs = out_specs[0]
        out_shape = out_shapes[0]
    else:
        out_shape = out_shapes
    return pl.pallas_call(
        body, grid=grid, in_specs=in_specs, out_specs=out_specs,
        out_shape=out_shape)(*ins)


def _k_proj(x_ref, w_ref, as_ref, ad_ref, p_ref, st_ref):
    x = x_ref[...]
    proj = jnp.dot(x, w_ref[...], preferred_element_type=_f32)
    ss = jnp.dot(proj, as_ref[...], preferred_element_type=_f32)
    p_ref[...] = jnp.concatenate(
        [proj, ss, jnp.zeros((BLK, 32), _f32)], axis=1)
    st_ref[...] = jnp.dot(proj, ad_ref[...], preferred_element_type=_f32)


def _k_scale(nrows_valid, g_ref, stdst_ref, eh_ref, sv_ref):
    i = pl.program_id(0)
    g = g_ref[...]
    s = g[:, DP:DP + 16] + stdst_ref[...]
    ex = jnp.exp(jnp.maximum(s, 0.2 * s))
    rows = i * BLK + lax.broadcasted_iota(jnp.int32, (BLK, 1), 0)
    lanes = lax.broadcasted_iota(jnp.int32, (BLK, 16), 1)
    ex = jnp.where((rows < nrows_valid) & (lanes < NH), ex, 0.0)
    scaled = g[:, :DP] * jnp.dot(ex, eh_ref[...], preferred_element_type=_f32)
    sv_ref[...] = jnp.concatenate([scaled, ex], axis=1)


def _k_epi(is_last, nrows_valid, acc_ref, x_ref, b_ref, eh_ref, *o_refs):
    acc = acc_ref[...]
    d = jnp.dot(acc[:, DP:], eh_ref[...], preferred_element_type=_f32)
    out = acc[:, :DP] / (d + 1e-16) + x_ref[..., :DP] + b_ref[...]
    if not is_last:
        o_refs[0][...] = jnp.where(
            out > 0, out, jnp.exp(jnp.minimum(out, 0.0)) - 1.0)
    else:
        i = pl.program_id(0)
        rows = i * BLK + lax.broadcasted_iota(jnp.int32, (BLK, 1), 0)
        cols = lax.broadcasted_iota(jnp.int32, (BLK, DP), 1)
        out = jnp.where(cols == D, 1.0, out)
        out = jnp.where(rows < nrows_valid, out, 0.0)
        o_refs[0][...] = out
        o_refs[1][...] = jnp.concatenate(
            [out, jnp.zeros((BLK, 48), _f32)], axis=1)


def _k_idx2(i_ref, lo_ref, hi_ref):
    ix = i_ref[...]
    tr = T2 + (ix & (TRASH - 1))
    lo_ref[...] = jnp.where(ix < T2, ix, tr)
    hi_ref[...] = jnp.where(ix >= T2, ix - T2, tr)


def _mk_idx2(idxp):
    m = idxp.shape[0]
    r = m // 128
    shp = jax.ShapeDtypeStruct((r, 128), jnp.int32)
    lo, hi = pl.pallas_call(
        _k_idx2, grid=(1,),
        in_specs=[pl.BlockSpec((r, 128), lambda i: (0, 0))],
        out_specs=[pl.BlockSpec((r, 128), lambda i: (0, 0))] * 2,
        out_shape=[shp, shp])(idxp.reshape(r, 128))
    return jnp.concatenate([lo.reshape(-1), hi.reshape(-1)])


def _k_out2norm(acc_ref, o_ref):
    acc = acc_ref[...]
    cnt = jnp.maximum(acc[:, D:D + 1], 1.0)
    cols = lax.broadcasted_iota(jnp.int32, (BLK, DP), 1)
    o_ref[...] = jnp.concatenate(
        [jnp.where(cols < D, acc / cnt, 0.0), jnp.zeros((BLK, 48), _f32)],
        axis=1)


def _k_wvals(ee_ref, z_ref, lwa_ref, lwb_ref, lb_ref, wv_ref):
    ee = ee_ref[..., :DP]
    z = z_ref[..., :DP]
    ne = (jnp.sum(ee * lwa_ref[...], axis=-1, keepdims=True)
          + jnp.sum(z * lwb_ref[...], axis=-1, keepdims=True) + lb_ref[...])
    ex2 = jnp.exp(ne)
    i = pl.program_id(0)
    rows = i * BLK + lax.broadcasted_iota(jnp.int32, (BLK, 1), 0)
    ex2 = jnp.where(rows < N_SUB, ex2, 0.0)
    wv_ref[...] = jnp.concatenate([ee * ex2, z * ex2], axis=1)


def _k_subrel(ee_ref, z_ref, wa_ref, wb_ref, b_ref, o_ref):
    srl = (jnp.dot(ee_ref[..., :DP], wa_ref[...], preferred_element_type=_f32)
           + jnp.dot(z_ref[..., :DP], wb_ref[...], preferred_element_type=_f32)
           + b_ref[...])
    o_ref[...] = jnp.concatenate([srl, jnp.zeros((BLK, 48), _f32)], axis=1)


def _k_head(acc_ref, w_ref, b_ref, ent_ref, o_ref):
    acc = acc_ref[...]
    srow = 1.0 / (acc[:, D:D + 1] + 1e-16)
    head = (jnp.dot(acc, w_ref[...], preferred_element_type=_f32) * srow
            + b_ref[...])
    o_ref[...] = (jnp.concatenate([head, jnp.zeros((BLK, 48), _f32)], axis=1)
                  + ent_ref[...])


def _k_ln(x_ref, g_ref, b_ref, o_ref, oc_ref):
    x = x_ref[...]
    mu = jnp.sum(x, axis=-1, keepdims=True) * (1.0 / D)
    v = jnp.sum(x * x, axis=-1, keepdims=True) * (1.0 / D) - mu * mu
    y = (x - mu) * lax.rsqrt(v + 1e-5) * g_ref[...] + b_ref[...]
    o_ref[...] = y
    oc_ref[...] = y[:, :D]


def _k_padent(x_ref, o_ref):
    o_ref[...] = jnp.concatenate(
        [x_ref[...], jnp.zeros((600, DG - D), _f32)], axis=1)


def _k_loss(ee_ref, sr_ref, see_ref, ssr_ref, o_ref):
    ee = ee_ref[...]
    sr = sr_ref[...]
    se = see_ref[...]
    ss = ssr_ref[...]
    laa = jax.nn.sigmoid(jnp.sum(ee * sr, axis=-1, keepdims=True))
    lbb = jax.nn.sigmoid(jnp.sum(se * ss, axis=-1, keepdims=True))
    lab = jax.nn.sigmoid(jnp.sum(ee * ss, axis=-1, keepdims=True))
    lba = jax.nn.sigmoid(jnp.sum(se * sr, axis=-1, keepdims=True))
    i = pl.program_id(0)
    rows = i * BLK + lax.broadcasted_iota(jnp.int32, (BLK, 1), 0)
    ok = rows < N_SUB
    p0 = jnp.sum(jnp.where(ok, jnp.maximum(lba - laa + 0.5, 0.0), 0.0))
    p1 = jnp.sum(jnp.where(ok, jnp.maximum(lab - lbb + 0.5, 0.0), 0.0))

    @pl.when(i == 0)
    def _():
        o_ref[...] = jnp.zeros((8, 128), _f32)

    lanes = lax.broadcasted_iota(jnp.int32, (8, 128), 1)
    rows8 = lax.broadcasted_iota(jnp.int32, (8, 128), 0)
    o_ref[...] += jnp.where(
        rows8 == 0, jnp.where(lanes == 0, p0, jnp.where(lanes == 1, p1, 0.0)),
        0.0)


def _k_count(b_ref, o_ref):
    blk = b_ref[...]
    lanes = lax.broadcasted_iota(jnp.int32, (1, 16), 1)
    acc = jnp.zeros((1, 16), _f32)
    for g in range(NUM_REL):
        cg = jnp.sum((blk == g).astype(_f32))
        acc = jnp.where(lanes == g, cg, acc)
    o_ref[...] = acc


def _k_relcon(rp_ref, m28_ref, w_ref, b2_ref, g_ref, b_ref, o_ref):
    rp = rp_ref[...]
    rows = lax.broadcasted_iota(jnp.int32, (32, 128), 0)
    low = jnp.where(rows < NUM_REL, rp, 0.0)
    tile14 = low + pltpu.roll(low, NUM_REL, 0)
    nr = tile14 * m28_ref[...] + rp
    y = jnp.dot(nr, w_ref[...], preferred_element_type=_f32) + b2_ref[...]
    mu = jnp.sum(y, axis=-1, keepdims=True) * (1.0 / D)
    v = jnp.sum(y * y, axis=-1, keepdims=True) * (1.0 / D) - mu * mu
    o_ref[...] = (y - mu) * lax.rsqrt(v + 1e-5) * g_ref[...] + b_ref[...]


def _k_relemb(id_ref, rc_ref, o_ref):
    oh = (id_ref[...] == lax.broadcasted_iota(jnp.int32, (BLK, 32), 1))
    o_ref[...] = jnp.dot(oh.astype(_f32), rc_ref[...],
                         preferred_element_type=_f32)


def _pad_rows(a, n):
    return jnp.pad(a, ((0, n - a.shape[0]),) + ((0, 0),) * (a.ndim - 1))


def _pad_idx(a, n):
    return jnp.pad(a.astype(jnp.int32), (0, n - a.shape[0]))


def _padw(w, r, c):
    return jnp.pad(w, ((0, r - w.shape[0]), (0, c - w.shape[1])))


def _gat_layer_pallas(xp, Wp, As16, Ad16, biasp, Eh, srcp, dstp, dst_i2, xw):
    P, stT = _rows_pc(
        _k_proj, N_PAD,
        [jax.ShapeDtypeStruct((N_PAD, DG), _f32),
         jax.ShapeDtypeStruct((N_PAD, 16), _f32)],
        [xp, Wp, As16, Ad16], [xw, None, None, None])
    stdst = _sc_gather(stT, dstp, 16, 2048)
    g = _sc_gather(P, srcp, DG, 384, tiled=True)
    sv = _rows_pc(
        functools.partial(_k_scale, E), E_PAD,
        [jax.ShapeDtypeStruct((E_PAD, DP + 16), _f32)],
        [g, stdst, Eh], [DG, 16, None])
    acc = _sc_scatter_add(sv, dst_i2, T2, DP + 16, 2048)
    return acc


def kernel(entity_emb, rel_param, W1, a_src1, a_dst1, bias1, W2, a_src2, a_dst2, bias2,
           layer_emb_W, layer_emb_b, rel_W, rel_b, sub_W, sub_b, out_W, out_b,
           ln_node_g, ln_node_b, ln_re_g, ln_re_b,
           b_x, edge_index, b_node_graph_index, sub, rel, shuf_index):
    entity_embp = pl.pallas_call(
        _k_padent, grid=(NUM_ENT // 600,),
        in_specs=[pl.BlockSpec((600, D), lambda i: (i, 0))],
        out_specs=pl.BlockSpec((600, DG), lambda i: (i, 0)),
        out_shape=jax.ShapeDtypeStruct((NUM_ENT, DG), _f32))(entity_emb)
    srcp = _pad_idx(edge_index[0], E_PAD)
    dstp = _pad_idx(edge_index[1], E_PAD)
    b_xp = _pad_idx(b_x, N_PAD)
    shufp = _pad_idx(shuf_index, N_PAD)
    subi = sub.astype(jnp.int32)

    hsel = jnp.repeat(jnp.arange(NH), FH)
    def _mk_a16(a):
        m = jnp.zeros((DP, 16), _f32)
        return m.at[jnp.arange(D), hsel].set(a.reshape(-1))
    As1, Ad1 = _mk_a16(a_src1), _mk_a16(a_dst1)
    As2, Ad2 = _mk_a16(a_src2), _mk_a16(a_dst2)
    Eh = jnp.zeros((16, DP), _f32).at[hsel, jnp.arange(D)].set(1.0)
    W1p = _padw(W1, DG, DP)
    W2p = _padw(W2, DP, DP)
    b1p = _padw(bias1.reshape(1, -1), 1, DP)
    b2p = _padw(bias2.reshape(1, -1), 1, DP)
    lwa = _padw(layer_emb_W[:D].reshape(1, -1), 1, DP)
    lwb = _padw(layer_emb_W[D:].reshape(1, -1), 1, DP)
    lb = layer_emb_b.reshape(1, 1)
    sWa = _padw(sub_W[:D], DP, DP)
    sWb = _padw(sub_W[D:], DP, DP)
    sbp = _padw(sub_b.reshape(1, -1), 1, DP)
    W6 = jnp.zeros((2 * DP, DP), _f32)
    W6 = W6.at[:D, :D].set(out_W[:D]).at[DP:DP + D, :D].set(out_W[D:])
    obp = _padw(out_b.reshape(1, -1), 1, DP)
    lngp = _padw(ln_node_g.reshape(1, -1), 1, DG)
    lnbp = _padw(ln_node_b.reshape(1, -1), 1, DG)
    lngr = _padw(ln_re_g.reshape(1, -1), 1, DP)
    lnbr = _padw(ln_re_b.reshape(1, -1), 1, DP)
    rWp = _padw(rel_W, 100, DP)
    rb2 = _padw((2.0 * rel_b).reshape(1, -1), 1, DP)
    bngi2 = jnp.pad(b_node_graph_index.astype(jnp.int32), (0, N_PAD - N_SUB),
                    constant_values=15).reshape(N_PAD // 128, 128)

    dst_i2 = _mk_idx2(dstp)
    bx_i2 = _mk_idx2(b_xp)

    xp = _sc_gather(entity_embp, b_xp, DG, 384, tiled=True)
    acc1 = _gat_layer_pallas(xp, W1p, As1, Ad1, b1p, Eh, srcp, dstp, dst_i2,
                             DG)
    h = _rows_pc(functools.partial(_k_epi, False, N_SUB), N_PAD,
                 [jax.ShapeDtypeStruct((N_PAD, DP), _f32)],
                 [acc1, xp, b1p, Eh], [DP + 16, DG, None, None])
    acc2 = _gat_layer_pallas(h, W2p, As2, Ad2, b2p, Eh, srcp, dstp, dst_i2,
                             DP)
    eep, eep256 = _rows_pc(functools.partial(_k_epi, True, N_SUB), N_PAD,
                           [jax.ShapeDtypeStruct((N_PAD, DP), _f32),
                            jax.ShapeDtypeStruct((N_PAD, DG), _f32)],
                           [acc2, h, b2p, Eh], [DP + 16, DP, None, None])

    accB = _sc_scatter_add(eep, bx_i2, T2, DP, 2048)
    out2 = _rows_pc(_k_out2norm, N_PAD,
                    [jax.ShapeDtypeStruct((N_PAD, DG), _f32)], [accB], [DP])
    z = _sc_gather(out2, b_xp, DG, 384, tiled=True)
    ee_bx = _sc_gather(eep256, b_xp, DG, 384, tiled=True)
    wv = _rows_pc(_k_wvals, N_PAD,
                  [jax.ShapeDtypeStruct((N_PAD, 2 * DP), _f32)],
                  [ee_bx, z, lwa, lwb, lb], [DG, DG, None, None, None])
    acc3 = _sc_scatter_add(wv, bx_i2, T2, 2 * DP, 2048)
    head = _rows_pc(_k_head, N_PAD,
                    [jax.ShapeDtypeStruct((N_PAD, DG), _f32)],
                    [acc3, W6, obp, entity_embp[:N_PAD]],
                    [2 * DP, None, None, DG])
    lnin = jnp.concatenate([head[:KSEG], entity_embp[KSEG:]], 0)
    entity_con_p, entity_con = pl.pallas_call(
        _k_ln, grid=(NUM_ENT // 600,),
        in_specs=[pl.BlockSpec((600, DG), lambda i: (i, 0)),
                  pl.BlockSpec((1, DG), lambda i: (0, 0)),
                  pl.BlockSpec((1, DG), lambda i: (0, 0))],
        out_specs=[pl.BlockSpec((600, DG), lambda i: (i, 0)),
                   pl.BlockSpec((600, D), lambda i: (i, 0))],
        out_shape=[jax.ShapeDtypeStruct((NUM_ENT, DG), _f32),
                   jax.ShapeDtypeStruct((NUM_ENT, D), _f32)])(
            lnin, lngp, lnbp)

    cnt = pl.pallas_call(
        _k_count, grid=(1,),
        in_specs=[pl.BlockSpec((N_PAD // 128, 128), lambda i: (0, 0))],
        out_specs=pl.BlockSpec((1, 16), lambda i: (0, 0)),
        out_shape=jax.ShapeDtypeStruct((1, 16), _f32))(bngi2)
    m14 = (cnt[0, :NUM_REL] > 0).astype(_f32)
    m28 = jnp.concatenate([m14, m14, jnp.zeros((4,), _f32)]).reshape(32, 1)
    rp32 = jnp.pad(rel_param, ((0, 4), (0, 28)))
    rW128 = jnp.pad(rWp, ((0, 28), (0, 0)))
    rc32 = pl.pallas_call(
        _k_relcon, grid=(1,),
        in_specs=[pl.BlockSpec((32, 128), lambda i: (0, 0)),
                  pl.BlockSpec((32, 1), lambda i: (0, 0)),
                  pl.BlockSpec((128, DP), lambda i: (0, 0)),
                  pl.BlockSpec((1, DP), lambda i: (0, 0)),
                  pl.BlockSpec((1, DP), lambda i: (0, 0)),
                  pl.BlockSpec((1, DP), lambda i: (0, 0))],
        out_specs=pl.BlockSpec((32, DP), lambda i: (0, 0)),
        out_shape=jax.ShapeDtypeStruct((32, DP), _f32))(
            rp32, m28, rW128, rb2, lngr, lnbr)
    rel_con = rc32[:2 * NUM_REL, :D]
    rel_emb = _rows_pc(_k_relemb, BQ,
                       [jax.ShapeDtypeStruct((BQ, DP), _f32)],
                       [rel.astype(jnp.int32).reshape(BQ, 1), rc32],
                       [1, None])[:, :D]

    srl = _rows_pc(_k_subrel, N_PAD,
                   [jax.ShapeDtypeStruct((N_PAD, DG), _f32)],
                   [ee_bx, z, sWa, sWb, sbp], [DG, DG, None, None, None])
    se_ = _sc_gather(eep256, shufp, DG, 384, tiled=True)
    sr_ = _sc_gather(srl, shufp, DG, 384, tiled=True)
    parts = pl.pallas_call(
        _k_loss, grid=(N_PAD // BLK,),
        in_specs=[pl.BlockSpec((BLK, DG), lambda i: (i, 0))] * 4,
        out_specs=pl.BlockSpec((8, 128), lambda i: (0, 0)),
        out_shape=jax.ShapeDtypeStruct((8, 128), _f32))(
            eep256, srl, se_, sr_)
    cl_loss = (parts[0, 0] + parts[0, 1]) / N_SUB

    sub_emb = _sc_gather(entity_con_p, subi, DG, 384, tiled=True)[:, :D]
    return (sub_emb, rel_emb, entity_con, cl_loss, rel_con)
